# Initial kernel scaffold; baseline (speedup 1.0000x reference)
import jax
import jax.numpy as jnp
from jax.experimental import pallas as pl


def kernel(x_e1, edge_index1, rel1, edge_index_all1, rel_all1, x_name1, onehot1, x_e2, edge_index2, rel2, edge_index_all2, rel_all2, x_name2, onehot2, data_batch, params):
    raise NotImplementedError("write your pallas kernel here")


# speedup vs baseline: 6.3715x; 6.3715x over previous
import functools
import jax
import jax.numpy as jnp
from jax import lax
from jax.experimental import pallas as pl
from jax.experimental.pallas import tpu as pltpu
from jax.experimental.pallas import tpu_sc as plsc

NC, NS = 2, 16
NP = 10240
E = 160000
NRELP = 512
W300 = 320
W600 = 640
W900 = 912
WH = 128

_SC_PARAMS = pltpu.CompilerParams(needs_layout_passes=False,
                                  use_tc_tiling_on_sc=False)
_MESH = dict(core_axis_name="c", subcore_axis_name="s", num_cores=NC,
             num_subcores=NS)


def _make_deg_pass():
    per_w = E // (NC * NS)
    mesh = plsc.VectorSubcoreMesh(**_MESH)
    scratch = dict(
        acc=pltpu.VMEM((NP,), jnp.float32),
        iv=pltpu.VMEM((per_w,), jnp.int32),
        obuf=pltpu.VMEM((NP // NS,), jnp.float32),
        rbuf=pltpu.VMEM((NS, NP // NS), jnp.float32),
        shr=pltpu.VMEM_SHARED((NS, NP), jnp.float32),
    )

    def body(idx, out, sc):
        c = lax.axis_index("c")
        s = lax.axis_index("s")
        w = s * NC + c
        z16 = jnp.zeros((16,), jnp.float32)
        ones = jnp.ones((16,), jnp.float32)
        miota = lax.iota(jnp.int32, 16)

        def zrow(r, _):
            sc["acc"][pl.ds(r * 16, 16)] = z16
            return 0

        lax.fori_loop(0, NP // 16, zrow, 0)
        pltpu.sync_copy(idx.at[pl.ds(w * per_w, per_w)], sc["iv"])

        def step(g, _):
            i16 = sc["iv"][pl.ds(g * 16, 16)]
            for l in range(16):
                plsc.addupdate_scatter(sc["acc"], [i16], ones,
                                       mask=miota == l)
            return 0

        lax.fori_loop(0, per_w // 16, step, 0)
        rem = per_w % 16
        if rem:
            i16 = sc["iv"][pl.ds(per_w - 16, 16)]
            for l in range(16 - rem, 16):
                plsc.addupdate_scatter(sc["acc"], [i16], ones,
                                       mask=miota == l)
        pltpu.sync_copy(sc["acc"], sc["shr"].at[s])
        plsc.subcore_barrier()
        cols = NP // NS
        pltpu.sync_copy(sc["shr"].at[:, pl.ds(s * cols, cols)], sc["rbuf"])
        for f in range(cols // 16):
            v = jnp.zeros((16,), jnp.float32)
            for r in range(NS):
                v = v + sc["rbuf"][r, pl.ds(f * 16, 16)]
            sc["obuf"][pl.ds(f * 16, 16)] = v
        pltpu.sync_copy(sc["obuf"], out.at[c].at[pl.ds(s * cols, cols)])

    return pl.kernel(body, out_type=jax.ShapeDtypeStruct((NC, NP),
                                                         jnp.float32),
                     mesh=mesh, scratch_types=[scratch],
                     compiler_params=_SC_PARAMS)


def _make_weight_pass(na, nb):
    per_w = E // (NC * NS)
    mesh = plsc.VectorSubcoreMesh(**_MESH)
    scratch = dict(
        ta=pltpu.VMEM((na,), jnp.float32),
        tb=pltpu.VMEM((nb,), jnp.float32),
        av=pltpu.VMEM((per_w,), jnp.int32),
        bv=pltpu.VMEM((per_w,), jnp.int32),
        wv=pltpu.VMEM((per_w,), jnp.float32),
        cv=pltpu.VMEM((16,), jnp.float32),
    )

    def body(tab_a, tab_b, idx_a, idx_b, cvec, out, sc):
        c = lax.axis_index("c")
        s = lax.axis_index("s")
        w = s * NC + c
        base = w * per_w
        pltpu.sync_copy(tab_a, sc["ta"])
        pltpu.sync_copy(tab_b, sc["tb"])
        pltpu.sync_copy(cvec, sc["cv"])
        pltpu.sync_copy(idx_a.at[pl.ds(base, per_w)], sc["av"])
        pltpu.sync_copy(idx_b.at[pl.ds(base, per_w)], sc["bv"])
        cval = sc["cv"][:]

        def compute(off):
            a16 = plsc.load_gather(sc["ta"], [sc["av"][pl.ds(off, 16)]])
            b16 = plsc.load_gather(sc["tb"], [sc["bv"][pl.ds(off, 16)]])
            e = a16 + b16
            t = jnp.where(e >= 0, e, 0.01 * e)
            sc["wv"][pl.ds(off, 16)] = jnp.exp(t - cval)

        def step(g, _):
            compute(g * 16)
            return 0

        lax.fori_loop(0, per_w // 16, step, 0)
        if per_w % 16:
            compute(per_w - 16)
        pltpu.sync_copy(sc["wv"], out.at[pl.ds(base, per_w)])

    return pl.kernel(body, out_type=jax.ShapeDtypeStruct((E,), jnp.float32),
                     mesh=mesh, scratch_types=[scratch],
                     compiler_params=_SC_PARAMS)


def _make_edge_pass(n_dst, fc, weighted):
    n_chunks = E // 128
    base_n, extra = divmod(n_chunks, NS)
    rows_t = n_dst // NS
    mesh = plsc.VectorSubcoreMesh(**_MESH)

    scratch = dict(
        acc=pltpu.VMEM_SHARED((n_dst, fc), jnp.float32),
        abuf=pltpu.VMEM((128, fc), jnp.float32),
        jv=pltpu.VMEM((128,), jnp.int32),
        iv=pltpu.VMEM((128,), jnp.int32),
        sem=pltpu.SemaphoreType.DMA,
    )
    if weighted:
        scratch["wv"] = pltpu.VMEM((144,), jnp.float32)

    def body(*refs):
        if weighted:
            (table, idx_g, idx_s, wref, out, sc) = refs
        else:
            (table, idx_g, idx_s, out, sc) = refs
        c = lax.axis_index("c")
        s = lax.axis_index("s")

        z16 = jnp.zeros((16,), jnp.float32)

        def zrow(r, _):
            for f in range(fc // 16):
                sc["abuf"][r, pl.ds(16 * f, 16)] = z16
            return 0

        lax.fori_loop(0, 128, zrow, 0)
        nz_full, nz_rem = divmod(rows_t, 128)
        for rchunk in range(nz_full):
            pltpu.sync_copy(
                sc["abuf"],
                sc["acc"].at[pl.ds(s * rows_t + rchunk * 128, 128), :])
        if nz_rem:
            pltpu.sync_copy(
                sc["abuf"].at[pl.ds(0, nz_rem), :],
                sc["acc"].at[pl.ds(s * rows_t + nz_full * 128, nz_rem), :])
        plsc.subcore_barrier()

        n_my = base_n + jnp.where(s < extra, 1, 0)

        def chunk_loop(kk, _):
            base = (s + kk * NS) * 128
            pltpu.sync_copy(idx_g.at[pl.ds(base, 128)], sc["jv"])
            pltpu.sync_copy(idx_s.at[pl.ds(base, 128)], sc["iv"])
            pltpu.async_copy(table.at[c].at[sc["jv"]], sc["abuf"],
                             sc["sem"]).wait()
            if weighted:
                pltpu.sync_copy(wref.at[pl.ds(base, 128)],
                                sc["wv"].at[pl.ds(0, 128)])

                def scale_row(r, _):
                    w = jnp.full((16,), sc["wv"][pl.ds(r, 16)][0],
                                 jnp.float32)
                    for f in range(fc // 16):
                        sc["abuf"][r, pl.ds(16 * f, 16)] = (
                            sc["abuf"][r, pl.ds(16 * f, 16)] * w)
                    return 0

                lax.fori_loop(0, 128, scale_row, 0)
            pltpu.async_copy(sc["abuf"], sc["acc"].at[sc["iv"]], sc["sem"],
                             add=True).wait()
            return 0

        lax.fori_loop(0, n_my, chunk_loop, 0)
        plsc.subcore_barrier()
        pltpu.sync_copy(sc["acc"].at[pl.ds(s * rows_t, rows_t), :],
                        out.at[c].at[pl.ds(s * rows_t, rows_t), :])

    return pl.kernel(body,
                     out_type=jax.ShapeDtypeStruct((NC, n_dst, fc),
                                                   jnp.float32),
                     mesh=mesh, scratch_types=[scratch],
                     compiler_params=_SC_PARAMS)


def _make_bgather():
    B = 256
    per_w = B // (NC * NS)
    mesh = plsc.VectorSubcoreMesh(**_MESH)
    scratch = dict(
        iv=pltpu.VMEM((per_w,), jnp.int32),
        rbuf=pltpu.VMEM((per_w, W900), jnp.float32),
        sem=pltpu.SemaphoreType.DMA,
    )

    def body(t0, t1, t2, idx, out, sc):
        c = lax.axis_index("c")
        s = lax.axis_index("s")
        w = s * NC + c
        pltpu.sync_copy(idx.at[pl.ds(w * per_w, per_w)], sc["iv"])
        for t, tab in enumerate((t0, t1, t2)):
            pltpu.async_copy(tab.at[sc["iv"]], sc["rbuf"], sc["sem"]).wait()
            pltpu.sync_copy(sc["rbuf"],
                            out.at[t].at[pl.ds(w * per_w, per_w), :])

    return pl.kernel(body,
                     out_type=jax.ShapeDtypeStruct((3, 256, W900),
                                                   jnp.float32),
                     mesh=mesh, scratch_types=[scratch],
                     compiler_params=_SC_PARAMS)


def _make_factor_pass():
    B = 256
    mesh = plsc.VectorSubcoreMesh(**_MESH)
    scratch = dict(
        fv=pltpu.VMEM((NP,), jnp.float32),
        iv=pltpu.VMEM((B,), jnp.int32),
        vv=pltpu.VMEM((B,), jnp.float32),
    )

    def body(idxs, vals, out, sc):
        c = lax.axis_index("c")
        s = lax.axis_index("s")
        w = s * NC + c
        miota = lax.iota(jnp.int32, 16)

        @pl.when(w < 4)
        def _():
            ones = jnp.ones((16,), jnp.float32)

            def orow(r, _):
                sc["fv"][pl.ds(r * 16, 16)] = ones
                return 0

            lax.fori_loop(0, NP // 16, orow, 0)
            pltpu.sync_copy(idxs.at[w // 2], sc["iv"])
            pltpu.sync_copy(vals.at[w], sc["vv"])

            def step(g, _):
                i16 = sc["iv"][pl.ds(g * 16, 16)]
                v16 = sc["vv"][pl.ds(g * 16, 16)]
                for l in range(16):
                    plsc.store_scatter(sc["fv"], [i16], v16,
                                       mask=miota == l)
                return 0

            lax.fori_loop(0, B // 16, step, 0)
            pltpu.sync_copy(sc["fv"], out.at[w])

    return pl.kernel(body, out_type=jax.ShapeDtypeStruct((4, NP),
                                                         jnp.float32),
                     mesh=mesh, scratch_types=[scratch],
                     compiler_params=_SC_PARAMS)


def _lrelu(x):
    return jnp.where(x >= 0, x, 0.01 * x)


def _t_dinv():
    def body(deg2, out):
        deg = deg2[0:1, :] + deg2[1:2, :]
        dinv = lax.rsqrt(deg)
        idx = lax.broadcasted_iota(jnp.int32, (1, NP), 1)
        dinv = jnp.where(idx < 10000, dinv, 0.0)
        out[...] = jnp.broadcast_to(dinv, (8, NP))

    return pl.pallas_call(body, out_shape=jax.ShapeDtypeStruct((8, NP),
                                                               jnp.float32))


def _t_prescale():
    def body(x, dinv, out):
        out[...] = x[...] * dinv[:, 0:1]

    return pl.pallas_call(body, out_shape=jax.ShapeDtypeStruct((NP, W300),
                                                               jnp.float32))


def _t_highway(prescale):
    blk = 2048
    grid = (NP // blk,)

    def body(x, graw, dinv, wt, out, yout=None):
        x2 = jnp.maximum(graw[...] * dinv[:, 0:1], 0.0)
        gate = _lrelu(jnp.dot(x[...], wt[...],
                              preferred_element_type=jnp.float32) + 1e-08)
        o = _lrelu(gate * x2 + (1.0 - gate) * x[...])
        out[...] = o
        if prescale:
            yout[...] = o * dinv[:, 0:1]

    rowspec = pl.BlockSpec((blk, W300), lambda b: (b, 0))
    if prescale:
        out_shape = [jax.ShapeDtypeStruct((NP, W300), jnp.float32),
                     jax.ShapeDtypeStruct((NP, W300), jnp.float32)]
        out_specs = [rowspec, rowspec]
    else:
        out_shape = jax.ShapeDtypeStruct((NP, W300), jnp.float32)
        out_specs = rowspec

    return pl.pallas_call(
        body, grid=grid,
        in_specs=[rowspec,
                  rowspec,
                  pl.BlockSpec((blk, 128), lambda b: (b, 0)),
                  pl.BlockSpec((W300, W300), lambda b: (0, 0))],
        out_specs=out_specs, out_shape=out_shape)


def _t_proj():
    def body(x, wh_t, wt_t, a2, b2, c2, xh_o, xt_o, scal_o, cmax_o):
        xv = x[...]
        xh = jnp.dot(xv, wh_t[...], preferred_element_type=jnp.float32)
        xt = jnp.dot(xv, wt_t[...], preferred_element_type=jnp.float32)
        scal = (jnp.dot(xh, a2[...], preferred_element_type=jnp.float32)
                + jnp.dot(xt, b2[...], preferred_element_type=jnp.float32)
                + jnp.dot(xv, c2[...], preferred_element_type=jnp.float32))
        col = lax.broadcasted_iota(jnp.int32, (NP, WH), 1)
        xh_o[...] = jnp.where(col == 100, 1.0, xh)
        xt_o[...] = jnp.where(col == 100, 1.0, xt)
        scal_o[...] = scal
        cmax_o[...] = jnp.broadcast_to(
            jnp.max(scal, axis=0, keepdims=True), (8, WH))

    return pl.pallas_call(
        body,
        out_shape=[jax.ShapeDtypeStruct((NP, WH), jnp.float32),
                   jax.ShapeDtypeStruct((NP, WH), jnp.float32),
                   jax.ShapeDtypeStruct((NP, WH), jnp.float32),
                   jax.ShapeDtypeStruct((8, WH), jnp.float32)])


def _t_rel_finish():
    def body(acca, accb, grmat, t2_o, cmax_o):
        sa = acca[:, 100:101] + 1e-16
        sb = accb[:, 100:101] + 1e-16
        xr = acca[...] / sa + accb[...] / sb
        t2 = jnp.dot(xr, grmat[...], preferred_element_type=jnp.float32)
        t2_o[...] = t2
        cmax_o[...] = jnp.broadcast_to(
            jnp.max(t2, axis=0, keepdims=True), (8, WH))

    return pl.pallas_call(
        body,
        out_shape=[jax.ShapeDtypeStruct((NRELP, WH), jnp.float32),
                   jax.ShapeDtypeStruct((8, WH), jnp.float32)])


def _t_uv():
    def body(xc, gmat, uv_o, cmax_o):
        uv = jnp.dot(xc[...], gmat[...], preferred_element_type=jnp.float32)
        uv_o[...] = uv
        cmax_o[...] = jnp.broadcast_to(
            jnp.max(uv, axis=0, keepdims=True), (8, WH))

    return pl.pallas_call(
        body,
        out_shape=[jax.ShapeDtypeStruct((NP, WH), jnp.float32),
                   jax.ShapeDtypeStruct((8, WH), jnp.float32)])


def _t_gat_finish():
    blk = 2048
    grid = (NP // blk,)

    def body(gacc, xc, out):
        s = gacc[:, 300:301] + 1e-16
        agg = _lrelu(gacc[:, 0:300] / s)
        o = jnp.concatenate(
            [9.0 * xc[:, 0:600], 9.0 * agg,
             jnp.zeros((blk, W900 - 900), jnp.float32)], axis=1)
        out[...] = o

    return pl.pallas_call(
        body, grid=grid,
        in_specs=[pl.BlockSpec((blk, W300), lambda b: (b, 0)),
                  pl.BlockSpec((blk, W600), lambda b: (b, 0))],
        out_specs=pl.BlockSpec((blk, W900), lambda b: (b, 0)),
        out_shape=jax.ShapeDtypeStruct((NP, W900), jnp.float32))


def _t_lstm1():
    blk = 1024
    grid = (NP // blk,)
    GW = 1024

    def cell(xt, h, c, wf, uf, bf):
        g = (jnp.dot(xt, wf, preferred_element_type=jnp.float32) + bf
             + jnp.dot(h, uf, preferred_element_type=jnp.float32))
        ii = jax.nn.sigmoid(g[:, 0:256])
        ff = jax.nn.sigmoid(g[:, 256:512])
        gg = jnp.tanh(g[:, 512:768])
        oo = jax.nn.sigmoid(g[:, 768:1024])
        c = ff * c + ii * gg
        h = oo * jnp.tanh(c)
        return h, c

    def body(xs, wff, uff, bff, wfr, ufr, bfr, out):
        z = jnp.zeros((blk, 256), jnp.float32)
        h, c = z, z
        for t in range(3):
            h, c = cell(xs[t], h, c, wff[...], uff[...], bff[...])
            out[0, t] = h
        h, c = z, z
        for t in (2, 1, 0):
            h, c = cell(xs[t], h, c, wfr[...], ufr[...], bfr[...])
            out[1, t] = h

    return pl.pallas_call(
        body, grid=grid,
        in_specs=[pl.BlockSpec((3, blk, W300), lambda b: (0, b, 0)),
                  pl.BlockSpec((W300, GW), lambda b: (0, 0)),
                  pl.BlockSpec((256, GW), lambda b: (0, 0)),
                  pl.BlockSpec((1, GW), lambda b: (0, 0)),
                  pl.BlockSpec((W300, GW), lambda b: (0, 0)),
                  pl.BlockSpec((256, GW), lambda b: (0, 0)),
                  pl.BlockSpec((1, GW), lambda b: (0, 0))],
        out_specs=pl.BlockSpec((2, 3, blk, 256), lambda b: (0, 0, b, 0)),
        out_shape=jax.ShapeDtypeStruct((2, 3, NP, 256), jnp.float32))


def _t_lstm2():
    blk = 1024
    grid = (NP // blk,)
    GW = 1536

    def body(xs, wf, uf, bf, out):
        z = jnp.zeros((blk, 384), jnp.float32)
        h, c = z, z
        for t in range(3):
            g = (jnp.dot(xs[t], wf[...], preferred_element_type=jnp.float32)
                 + bf[...]
                 + jnp.dot(h, uf[...], preferred_element_type=jnp.float32))
            ii = jax.nn.sigmoid(g[:, 0:384])
            ff = jax.nn.sigmoid(g[:, 384:768])
            gg = jnp.tanh(g[:, 768:1152])
            oo = jax.nn.sigmoid(g[:, 1152:1536])
            c = ff * c + ii * gg
            h = oo * jnp.tanh(c)
            out[t] = h

    return pl.pallas_call(
        body, grid=grid,
        in_specs=[pl.BlockSpec((3, blk, W300), lambda b: (0, b, 0)),
                  pl.BlockSpec((W300, GW), lambda b: (0, 0)),
                  pl.BlockSpec((384, GW), lambda b: (0, 0)),
                  pl.BlockSpec((1, GW), lambda b: (0, 0))],
        out_specs=pl.BlockSpec((3, blk, 384), lambda b: (0, b, 0)),
        out_shape=jax.ShapeDtypeStruct((3, NP, 384), jnp.float32))


def _t_attn():
    nblk = 512
    grid = (3, NP // nblk)

    def body(q, kt, v, out, m_s, l_s, acc_s):
        n = pl.program_id(1)

        @pl.when(n == 0)
        def _():
            m_s[...] = jnp.full((256, 128), -1e30, jnp.float32)
            l_s[...] = jnp.zeros((256, 128), jnp.float32)
            acc_s[...] = jnp.zeros((256, W900), jnp.float32)

        s = jnp.dot(q[0], kt[0], preferred_element_type=jnp.float32)
        col = lax.broadcasted_iota(jnp.int32, (256, nblk), 1) + n * nblk
        s = jnp.where(col < 10000, s, -1e30)
        m_prev = m_s[:, 0:1]
        m_new = jnp.maximum(m_prev, jnp.max(s, axis=1, keepdims=True))
        p = jnp.exp(s - m_new)
        corr = jnp.exp(m_prev - m_new)
        l_new = l_s[:, 0:1] * corr + jnp.sum(p, axis=1, keepdims=True)
        acc_s[...] = acc_s[...] * corr + jnp.dot(
            p, v[0], preferred_element_type=jnp.float32)
        m_s[...] = jnp.broadcast_to(m_new, (256, 128))
        l_s[...] = jnp.broadcast_to(l_new, (256, 128))

        @pl.when(n == grid[1] - 1)
        def _():
            out[0] = acc_s[...] / l_s[:, 0:1] + q[0]

    return pl.pallas_call(
        body, grid=grid,
        in_specs=[pl.BlockSpec((1, 256, W900), lambda c, n: (c, 0, 0)),
                  pl.BlockSpec((1, W900, nblk), lambda c, n: (c, 0, n)),
                  pl.BlockSpec((1, nblk, W900), lambda c, n: (c, n, 0))],
        out_specs=pl.BlockSpec((1, 256, W900), lambda c, n: (c, 0, 0)),
        out_shape=jax.ShapeDtypeStruct((3, 256, W900), jnp.float32),
        scratch_shapes=[pltpu.VMEM((256, 128), jnp.float32),
                        pltpu.VMEM((256, 128), jnp.float32),
                        pltpu.VMEM((256, W900), jnp.float32)])


def _t_mk():
    def body(re_all, w1t, b1, w2t, b2, out):
        for g in range(2):
            zs = []
            for ch in range(3):
                h = jnp.maximum(
                    jnp.dot(re_all[g, ch], w1t[...],
                            preferred_element_type=jnp.float32) + b1[...],
                    0.0)
                z = jax.nn.sigmoid(
                    jnp.dot(h, w2t[...],
                            preferred_element_type=jnp.float32) + b2[...])
                zs.append(z[:, 0:1])
            m = jnp.maximum(jnp.maximum(zs[0], zs[1]), zs[2])
            es = [jnp.exp(z - m) for z in zs]
            tot = es[0] + es[1] + es[2]
            out[g, 0] = jnp.broadcast_to(es[1] / tot, (256, 128))
            out[g, 1] = jnp.broadcast_to(es[2] / tot, (256, 128))

    return pl.pallas_call(
        body,
        out_shape=jax.ShapeDtypeStruct((2, 2, 256, 128), jnp.float32))


def _t_final_mul():
    blk = 2048
    grid = (4, NP // blk)

    def body(x, f, out):
        out[...] = x[...] * f[:, :, 0:1]

    return pl.pallas_call(
        body, grid=grid,
        in_specs=[pl.BlockSpec((1, blk, W900), lambda g, b: (g, b, 0)),
                  pl.BlockSpec((1, blk, 128), lambda g, b: (g, b, 0))],
        out_specs=pl.BlockSpec((1, blk, W900), lambda g, b: (g, b, 0)),
        out_shape=jax.ShapeDtypeStruct((4, NP, W900), jnp.float32))


def _pad2(a, rows, cols):
    return jnp.pad(a, ((0, rows - a.shape[0]), (0, cols - a.shape[1])))


def _pad1(a, n):
    return jnp.pad(a, (0, n - a.shape[0]))


def _split_tab(a):
    f = a.shape[1] // 2
    return jnp.stack([a[:, :f], a[:, f:]])


def _packT(w, rows, cols, col_map):
    m = jnp.zeros((rows, cols), jnp.float32)
    for mat, off in col_map:
        m = m.at[:mat.shape[1], off:off + mat.shape[0]].set(mat.T)
    return m


def _cvec(x):
    return jnp.full((16,), x, jnp.float32)


def _branch(p, x_e, ei, rel, eia, x_name, wk):
    j_all, i_all = eia[0], eia[1]
    h_e, t_e = ei[0], ei[1]

    deg2 = wk["deg"](i_all)
    dinv = wk["dinv"](deg2)
    dinv128 = jnp.broadcast_to(dinv[0][:, None], (NP, 128))

    xp = _pad2(x_e, NP, W300)
    y = wk["prescale"](xp, dinv128)
    graw = wk["edge320u"](_split_tab(y), j_all, i_all)
    graw = jnp.concatenate([graw[0], graw[1]], axis=1)
    x1, y1 = wk["highway_p"](xp, graw, dinv128, wk["hw1T"])
    graw = wk["edge320u"](_split_tab(y1), j_all, i_all)
    graw = jnp.concatenate([graw[0], graw[1]], axis=1)
    x2 = wk["highway"](x1, graw, dinv128, wk["hw2T"])

    xh_aug, xt_aug, scal, cmax = wk["proj"](x2, wk["whT"], wk["wtT"],
                                            wk["a2"], wk["b2"], wk["c2"])
    pcol = jnp.asarray(scal[:, 0]).copy()
    qcol = jnp.asarray(scal[:, 1]).copy()
    rcol = jnp.asarray(scal[:, 2]).copy()
    s2col = jnp.asarray(scal[:, 3]).copy()
    ehcol = jnp.asarray(scal[:, 4]).copy()
    c1 = jnp.maximum(cmax[0, 0] + cmax[0, 1], 0.0)
    c2c = jnp.maximum(cmax[0, 2] + cmax[0, 3], 0.0)
    wA = wk["wpass_nn"](pcol, qcol, h_e, t_e, _cvec(c1))
    accA = wk["edge128r"](_split_tab(xh_aug), h_e, rel, wA)
    accA = jnp.concatenate([accA[0], accA[1]], axis=1)
    wB = wk["wpass_nn"](rcol, s2col, h_e, t_e, _cvec(c2c))
    accB = wk["edge128r"](_split_tab(xt_aug), t_e, rel, wB)
    accB = jnp.concatenate([accB[0], accB[1]], axis=1)
    t2, cmax_t2 = wk["rel_finish"](accA, accB, wk["grT"])
    t2col = jnp.asarray(t2[:, 0]).copy()

    cr = jnp.maximum(cmax[0, 4] + cmax_t2[0, 0], 0.0)
    wR = wk["wpass_nr"](ehcol, t2col, h_e, rel, _cvec(cr))
    x2_aug = jnp.concatenate(
        [x2[:, 0:300],
         jnp.ones((NP, 1), jnp.float32),
         jnp.zeros((NP, W300 - 301), jnp.float32)], axis=1)
    accR = wk["edge320r"](_split_tab(x2_aug), t_e, rel, wR)
    accR = jnp.concatenate([accR[0], accR[1]], axis=1)
    sR = accR[:, 300:301] + 1e-16
    gat_r_rows = accR / sR
    gat_r_full = jnp.zeros((NP, 300), jnp.float32).at[:NRELP].set(
        gat_r_rows[:, 0:300])

    x_cat = jnp.concatenate(
        [x2[:, 0:300], gat_r_full,
         jnp.zeros((NP, W600 - 600), jnp.float32)], axis=1)
    uv, cmax_uv = wk["uv"](x_cat, wk["gmat"])
    ucol = jnp.asarray(uv[:, 0]).copy()
    vcol = jnp.asarray(uv[:, 1]).copy()
    cg = jnp.maximum(cmax_uv[0, 0] + cmax_uv[0, 1], 0.0)
    wG = wk["wpass_nn"](ucol, vcol, i_all, j_all, _cvec(cg))
    accG = wk["edge320"](_split_tab(x2_aug), j_all, i_all, wG)
    accG = jnp.concatenate([accG[0], accG[1]], axis=1)
    xf = wk["gat_finish"](accG, x_cat)

    xs = jnp.stack([_pad2(x_name[:, 300 * t:300 * (t + 1)], NP, W300)
                    for t in range(3)])
    return xf, xs


def kernel(x_e1, edge_index1, rel1, edge_index_all1, rel_all1, x_name1,
           onehot1, x_e2, edge_index2, rel2, edge_index_all2, rel_all2,
           x_name2, onehot2, data_batch, params):
    p = params
    wk = {}
    wk["deg"] = _make_deg_pass()
    wk["dinv"] = _t_dinv()
    wk["prescale"] = _t_prescale()
    wk["edge320"] = _make_edge_pass(NP, 160, True)
    wk["edge320u"] = _make_edge_pass(NP, 160, False)
    wk["edge128r"] = _make_edge_pass(NRELP, 64, True)
    wk["edge320r"] = _make_edge_pass(NRELP, 160, True)
    wk["wpass_nn"] = _make_weight_pass(NP, NP)
    wk["wpass_nr"] = _make_weight_pass(NP, NRELP)
    wk["highway_p"] = _t_highway(True)
    wk["highway"] = _t_highway(False)
    wk["proj"] = _t_proj()
    wk["rel_finish"] = _t_rel_finish()
    wk["uv"] = _t_uv()
    wk["gat_finish"] = _t_gat_finish()
    wk["lstm1"] = _t_lstm1()
    wk["lstm2"] = _t_lstm2()
    wk["attn"] = _t_attn()
    wk["mk"] = _t_mk()
    wk["fmul"] = _t_final_mul()
    wk["bgather"] = _make_bgather()
    wk["factor"] = _make_factor_pass()

    wk["hw1T"] = _pad2(p["hw1_W"].T, W300, W300)
    wk["hw2T"] = _pad2(p["hw2_W"].T, W300, W300)
    wk["whT"] = _pad2(p["ge_wh"].T, W300, WH)
    wk["wtT"] = _pad2(p["ge_wt"].T, W300, WH)
    wk["a2"] = _packT(None, WH, WH, [(p["ge_ah1"], 0), (p["ge_at1"], 2)])
    wk["b2"] = _packT(None, WH, WH, [(p["ge_ah2"], 1), (p["ge_at2"], 3)])
    wk["c2"] = _packT(None, W300, WH, [(p["gr_ah"], 4)])
    wk["grT"] = _packT(None, WH, WH, [(p["gr_ar"], 0)])
    wk["gmat"] = _packT(None, W600, WH, [(p["g_ai"], 0), (p["g_aj"], 1)])

    def lstm1_w(pre):
        w = jnp.zeros((W300, 1024), jnp.float32)
        u = jnp.zeros((256, 1024), jnp.float32)
        b = jnp.zeros((1, 1024), jnp.float32)
        wih, whh = p[pre + "_Wih"], p[pre + "_Whh"]
        bih, bhh = p[pre + "_bih"], p[pre + "_bhh"]
        for g in range(4):
            w = w.at[:300, 256 * g:256 * g + 150].set(
                wih[150 * g:150 * (g + 1)].T)
            u = u.at[:150, 256 * g:256 * g + 150].set(
                whh[150 * g:150 * (g + 1)].T)
            b = b.at[0, 256 * g:256 * g + 150].set(
                bih[150 * g:150 * (g + 1)] + bhh[150 * g:150 * (g + 1)])
        return w, u, b

    l1f = lstm1_w("l1f")
    l1r = lstm1_w("l1r")

    w2 = jnp.zeros((W300, 1536), jnp.float32)
    u2 = jnp.zeros((384, 1536), jnp.float32)
    b2l = jnp.zeros((1, 1536), jnp.float32)
    for g in range(4):
        w2 = w2.at[:300, 384 * g:384 * g + 300].set(
            p["l2_Wih"][300 * g:300 * (g + 1)].T)
        u2 = u2.at[:300, 384 * g:384 * g + 300].set(
            p["l2_Whh"][300 * g:300 * (g + 1)].T)
        b2l = b2l.at[0, 384 * g:384 * g + 300].set(
            p["l2_bih"][300 * g:300 * (g + 1)]
            + p["l2_bhh"][300 * g:300 * (g + 1)])

    xf1, xs1 = _branch(p, x_e1, edge_index1, rel1, edge_index_all1, x_name1,
                       wk)
    xf2, xs2 = _branch(p, x_e2, edge_index2, rel2, edge_index_all2, x_name2,
                       wk)

    def lstms(xs, onehot):
        o1raw = wk["lstm1"](xs, *l1f, *l1r)
        o1 = jnp.concatenate(
            [o1raw[d, t][:, :150] for t in range(3) for d in range(2)],
            axis=1)
        o1 = _pad2(o1, NP, W900)
        xo = jnp.stack([_pad2(onehot[:, 300 * t:300 * (t + 1)], NP, W300)
                        for t in range(3)])
        o2raw = wk["lstm2"](xo, w2, u2, b2l)
        o2 = jnp.concatenate([o2raw[t][:, :300] for t in range(3)], axis=1)
        o2 = _pad2(o2, NP, W900)
        return o1, o2

    o11, o12 = lstms(xs1, onehot1)
    o21, o22 = lstms(xs2, onehot2)

    b0 = jnp.asarray(data_batch[:, 0]).copy()
    b1 = jnp.asarray(data_batch[:, 1]).copy()
    q1 = wk["bgather"](xf1, o11, o12, b0)
    q2 = wk["bgather"](xf2, o21, o22, b1)

    kv1 = jnp.stack([xf2, o21, o22])
    kv2 = jnp.stack([xf1, o11, o12])
    kt1 = jnp.transpose(kv1, (0, 2, 1))
    kt2 = jnp.transpose(kv2, (0, 2, 1))
    re1 = wk["attn"](q1, kt1, kv1)
    re2 = wk["attn"](q2, kt2, kv2)

    mk_w1t = _pad2(p["mk_W1"].T, W900, 512)
    mk_b1 = _pad1(p["mk_b1"], 512)[None, :]
    mk_w2t = _pad2(p["mk_W2"].T, 512, 128)
    mk_b2 = _pad1(p["mk_b2"], 128)[None, :]
    kg = wk["mk"](jnp.stack([re1, re2]), mk_w1t, mk_b1, mk_w2t, mk_b2)
    vals = jnp.stack([kg[0, 0, :, 0], kg[0, 1, :, 0],
                      kg[1, 0, :, 0], kg[1, 1, :, 0]])
    factors = wk["factor"](jnp.stack([b0, b1]), vals)
    fb = jnp.broadcast_to(factors[:, :, None], (4, NP, 128))
    stack4 = jnp.stack([o11, o12, o21, o22])
    mod = wk["fmul"](stack4, fb)

    def cut(a):
        return a[:10000, :900]

    return (cut(xf1), cut(mod[0]), cut(mod[1]),
            cut(xf2), cut(mod[2]), cut(mod[3]))


# speedup vs baseline: 7.3252x; 1.1497x over previous
import functools
import jax
import jax.numpy as jnp
from jax import lax
from jax.experimental import pallas as pl
from jax.experimental.pallas import tpu as pltpu
from jax.experimental.pallas import tpu_sc as plsc

NC, NS = 2, 16
NP = 10240
E = 160000
NRELP = 512
W300 = 320
W600 = 640
W900 = 912
WH = 128

_SC_PARAMS = pltpu.CompilerParams(needs_layout_passes=False,
                                  use_tc_tiling_on_sc=False)
_MESH = dict(core_axis_name="c", subcore_axis_name="s", num_cores=NC,
             num_subcores=NS)


def _make_deg_pass():
    per_w = E // (NC * NS)
    mesh = plsc.VectorSubcoreMesh(**_MESH)
    scratch = dict(
        acc=pltpu.VMEM((NP,), jnp.float32),
        iv=pltpu.VMEM((per_w,), jnp.int32),
        obuf=pltpu.VMEM((NP // NS,), jnp.float32),
        rbuf=pltpu.VMEM((NS, NP // NS), jnp.float32),
        shr=pltpu.VMEM_SHARED((NS, NP), jnp.float32),
    )

    def body(idx, out, sc):
        c = lax.axis_index("c")
        s = lax.axis_index("s")
        w = s * NC + c
        z16 = jnp.zeros((16,), jnp.float32)
        ones = jnp.ones((16,), jnp.float32)
        miota = lax.iota(jnp.int32, 16)

        def zrow(r, _):
            sc["acc"][pl.ds(r * 16, 16)] = z16
            return 0

        lax.fori_loop(0, NP // 16, zrow, 0)
        pltpu.sync_copy(idx.at[pl.ds(w * per_w, per_w)], sc["iv"])

        def step(g, _):
            i16 = sc["iv"][pl.ds(g * 16, 16)]
            for l in range(16):
                plsc.addupdate_scatter(sc["acc"], [i16], ones,
                                       mask=miota == l)
            return 0

        lax.fori_loop(0, per_w // 16, step, 0)
        rem = per_w % 16
        if rem:
            i16 = sc["iv"][pl.ds(per_w - 16, 16)]
            for l in range(16 - rem, 16):
                plsc.addupdate_scatter(sc["acc"], [i16], ones,
                                       mask=miota == l)
        pltpu.sync_copy(sc["acc"], sc["shr"].at[s])
        plsc.subcore_barrier()
        cols = NP // NS
        pltpu.sync_copy(sc["shr"].at[:, pl.ds(s * cols, cols)], sc["rbuf"])
        for f in range(cols // 16):
            v = jnp.zeros((16,), jnp.float32)
            for r in range(NS):
                v = v + sc["rbuf"][r, pl.ds(f * 16, 16)]
            sc["obuf"][pl.ds(f * 16, 16)] = v
        pltpu.sync_copy(sc["obuf"], out.at[c].at[pl.ds(s * cols, cols)])

    return pl.kernel(body, out_type=jax.ShapeDtypeStruct((NC, NP),
                                                         jnp.float32),
                     mesh=mesh, scratch_types=[scratch],
                     compiler_params=_SC_PARAMS)


def _make_weight_pass(na, nb):
    per_w = E // (NC * NS)
    mesh = plsc.VectorSubcoreMesh(**_MESH)
    scratch = dict(
        ta=pltpu.VMEM((na,), jnp.float32),
        tb=pltpu.VMEM((nb,), jnp.float32),
        av=pltpu.VMEM((per_w,), jnp.int32),
        bv=pltpu.VMEM((per_w,), jnp.int32),
        wv=pltpu.VMEM((per_w,), jnp.float32),
        cv=pltpu.VMEM((16,), jnp.float32),
    )

    def body(tab_a, tab_b, idx_a, idx_b, cvec, out, sc):
        c = lax.axis_index("c")
        s = lax.axis_index("s")
        w = s * NC + c
        base = w * per_w
        pltpu.sync_copy(tab_a, sc["ta"])
        pltpu.sync_copy(tab_b, sc["tb"])
        pltpu.sync_copy(cvec, sc["cv"])
        pltpu.sync_copy(idx_a.at[pl.ds(base, per_w)], sc["av"])
        pltpu.sync_copy(idx_b.at[pl.ds(base, per_w)], sc["bv"])
        cval = sc["cv"][:]

        def compute(off):
            a16 = plsc.load_gather(sc["ta"], [sc["av"][pl.ds(off, 16)]])
            b16 = plsc.load_gather(sc["tb"], [sc["bv"][pl.ds(off, 16)]])
            e = a16 + b16
            t = jnp.where(e >= 0, e, 0.01 * e)
            sc["wv"][pl.ds(off, 16)] = jnp.exp(t - cval)

        def step(g, _):
            compute(g * 16)
            return 0

        lax.fori_loop(0, per_w // 16, step, 0)
        if per_w % 16:
            compute(per_w - 16)
        pltpu.sync_copy(sc["wv"], out.at[pl.ds(base, per_w)])

    return pl.kernel(body, out_type=jax.ShapeDtypeStruct((E,), jnp.float32),
                     mesh=mesh, scratch_types=[scratch],
                     compiler_params=_SC_PARAMS)


def _make_edge_pass(n_dst, fc, weighted):
    n_sc = E // 1280
    base_n, extra = divmod(n_sc, NS)
    rows_t = n_dst // NS
    mesh = plsc.VectorSubcoreMesh(**_MESH)

    scratch = dict(
        acc=pltpu.VMEM_SHARED((n_dst, fc), jnp.float32),
        ab0=pltpu.VMEM((64, fc), jnp.float32),
        ab1=pltpu.VMEM((64, fc), jnp.float32),
        jv=pltpu.VMEM((20, 64), jnp.int32),
        iv=pltpu.VMEM((20, 64), jnp.int32),
        g0=pltpu.SemaphoreType.DMA,
        g1=pltpu.SemaphoreType.DMA,
        s0=pltpu.SemaphoreType.DMA,
        s1=pltpu.SemaphoreType.DMA,
    )
    if weighted:
        scratch["wv"] = pltpu.VMEM((1296,), jnp.float32)

    def body(*refs):
        if weighted:
            (table, idx_g, idx_s, wref, out, sc) = refs
        else:
            (table, idx_g, idx_s, out, sc) = refs
        c = lax.axis_index("c")
        s = lax.axis_index("s")

        z16 = jnp.zeros((16,), jnp.float32)

        def zrow(r, _):
            for f in range(fc // 16):
                sc["ab0"][r, pl.ds(16 * f, 16)] = z16
            return 0

        lax.fori_loop(0, 64, zrow, 0)
        nz_full, nz_rem = divmod(rows_t, 64)
        for rchunk in range(nz_full):
            pltpu.sync_copy(
                sc["ab0"],
                sc["acc"].at[pl.ds(s * rows_t + rchunk * 64, 64), :])
        if nz_rem:
            pltpu.sync_copy(
                sc["ab0"].at[pl.ds(0, nz_rem), :],
                sc["acc"].at[pl.ds(s * rows_t + nz_full * 64, nz_rem), :])
        plsc.subcore_barrier()

        n_my = base_n + jnp.where(s < extra, 1, 0)
        start = s * base_n + jnp.minimum(s, extra)

        def scale(buf, woff, _r0=None):
            def scale_row(r, _):
                w = jnp.full((16,), sc["wv"][pl.ds(woff + r, 16)][0],
                             jnp.float32)
                for f in range(fc // 16):
                    buf[r, pl.ds(16 * f, 16)] = (
                        buf[r, pl.ds(16 * f, 16)] * w)
                return 0

            lax.fori_loop(0, 64, scale_row, 0)

        def sc_loop(u, _):
            srow = (start + u) * 20
            sbase = (start + u) * 1280
            pltpu.sync_copy(idx_g.at[pl.ds(srow, 20), :], sc["jv"])
            pltpu.sync_copy(idx_s.at[pl.ds(srow, 20), :], sc["iv"])
            if weighted:
                pltpu.sync_copy(wref.at[pl.ds(sbase, 1280)],
                                sc["wv"].at[pl.ds(0, 1280)])

            def pair(t, _):
                ga = pltpu.async_copy(table.at[c].at[sc["jv"].at[2 * t]],
                                      sc["ab0"], sc["g0"])
                gb = pltpu.async_copy(table.at[c].at[sc["jv"].at[2 * t + 1]],
                                      sc["ab1"], sc["g1"])
                ga.wait()
                if weighted:
                    scale(sc["ab0"], 128 * t)
                sa = pltpu.async_copy(sc["ab0"],
                                      sc["acc"].at[sc["iv"].at[2 * t]],
                                      sc["s0"], add=True)
                gb.wait()
                if weighted:
                    scale(sc["ab1"], 128 * t + 64)
                sb = pltpu.async_copy(sc["ab1"],
                                      sc["acc"].at[sc["iv"].at[2 * t + 1]],
                                      sc["s1"], add=True)
                sa.wait()
                sb.wait()
                return 0

            lax.fori_loop(0, 10, pair, 0)
            return 0

        lax.fori_loop(0, n_my, sc_loop, 0)
        plsc.subcore_barrier()
        pltpu.sync_copy(sc["acc"].at[pl.ds(s * rows_t, rows_t), :],
                        out.at[c].at[pl.ds(s * rows_t, rows_t), :])

    return pl.kernel(body,
                     out_type=jax.ShapeDtypeStruct((NC, n_dst, fc),
                                                   jnp.float32),
                     mesh=mesh, scratch_types=[scratch],
                     compiler_params=_SC_PARAMS)


def _make_bgather():
    B = 256
    per_w = B // (NC * NS)
    mesh = plsc.VectorSubcoreMesh(**_MESH)
    scratch = dict(
        iv=pltpu.VMEM((per_w,), jnp.int32),
        rbuf=pltpu.VMEM((per_w, W900), jnp.float32),
        sem=pltpu.SemaphoreType.DMA,
    )

    def body(t0, t1, t2, idx, out, sc):
        c = lax.axis_index("c")
        s = lax.axis_index("s")
        w = s * NC + c
        pltpu.sync_copy(idx.at[pl.ds(w * per_w, per_w)], sc["iv"])
        for t, tab in enumerate((t0, t1, t2)):
            pltpu.async_copy(tab.at[sc["iv"]], sc["rbuf"], sc["sem"]).wait()
            pltpu.sync_copy(sc["rbuf"],
                            out.at[t].at[pl.ds(w * per_w, per_w), :])

    return pl.kernel(body,
                     out_type=jax.ShapeDtypeStruct((3, 256, W900),
                                                   jnp.float32),
                     mesh=mesh, scratch_types=[scratch],
                     compiler_params=_SC_PARAMS)


def _make_factor_pass():
    B = 256
    mesh = plsc.VectorSubcoreMesh(**_MESH)
    scratch = dict(
        fv=pltpu.VMEM((NP,), jnp.float32),
        iv=pltpu.VMEM((B,), jnp.int32),
        vv=pltpu.VMEM((B,), jnp.float32),
    )

    def body(idxs, vals, out, sc):
        c = lax.axis_index("c")
        s = lax.axis_index("s")
        w = s * NC + c
        miota = lax.iota(jnp.int32, 16)

        @pl.when(w < 4)
        def _():
            ones = jnp.ones((16,), jnp.float32)

            def orow(r, _):
                sc["fv"][pl.ds(r * 16, 16)] = ones
                return 0

            lax.fori_loop(0, NP // 16, orow, 0)
            pltpu.sync_copy(idxs.at[w // 2], sc["iv"])
            pltpu.sync_copy(vals.at[w], sc["vv"])

            def step(g, _):
                i16 = sc["iv"][pl.ds(g * 16, 16)]
                v16 = sc["vv"][pl.ds(g * 16, 16)]
                for l in range(16):
                    plsc.store_scatter(sc["fv"], [i16], v16,
                                       mask=miota == l)
                return 0

            lax.fori_loop(0, B // 16, step, 0)
            pltpu.sync_copy(sc["fv"], out.at[w])

    return pl.kernel(body, out_type=jax.ShapeDtypeStruct((4, NP),
                                                         jnp.float32),
                     mesh=mesh, scratch_types=[scratch],
                     compiler_params=_SC_PARAMS)


def _lrelu(x):
    return jnp.where(x >= 0, x, 0.01 * x)


def _t_dinv():
    def body(deg2, out):
        deg = deg2[0:1, :] + deg2[1:2, :]
        dinv = lax.rsqrt(deg)
        idx = lax.broadcasted_iota(jnp.int32, (1, NP), 1)
        dinv = jnp.where(idx < 10000, dinv, 0.0)
        out[...] = jnp.broadcast_to(dinv, (8, NP))

    return pl.pallas_call(body, out_shape=jax.ShapeDtypeStruct((8, NP),
                                                               jnp.float32))


def _t_prescale():
    def body(x, dinv, out):
        out[...] = x[...] * dinv[:, 0:1]

    return pl.pallas_call(body, out_shape=jax.ShapeDtypeStruct((NP, W300),
                                                               jnp.float32))


def _t_highway(prescale):
    blk = 2048
    grid = (NP // blk,)

    def body(x, graw, dinv, wt, out, yout=None):
        x2 = jnp.maximum(graw[...] * dinv[:, 0:1], 0.0)
        gate = _lrelu(jnp.dot(x[...], wt[...],
                              preferred_element_type=jnp.float32) + 1e-08)
        o = _lrelu(gate * x2 + (1.0 - gate) * x[...])
        out[...] = o
        if prescale:
            yout[...] = o * dinv[:, 0:1]

    rowspec = pl.BlockSpec((blk, W300), lambda b: (b, 0))
    if prescale:
        out_shape = [jax.ShapeDtypeStruct((NP, W300), jnp.float32),
                     jax.ShapeDtypeStruct((NP, W300), jnp.float32)]
        out_specs = [rowspec, rowspec]
    else:
        out_shape = jax.ShapeDtypeStruct((NP, W300), jnp.float32)
        out_specs = rowspec

    return pl.pallas_call(
        body, grid=grid,
        in_specs=[rowspec,
                  rowspec,
                  pl.BlockSpec((blk, 128), lambda b: (b, 0)),
                  pl.BlockSpec((W300, W300), lambda b: (0, 0))],
        out_specs=out_specs, out_shape=out_shape)


def _t_proj():
    def body(x, wh_t, wt_t, a2, b2, c2, xh_o, xt_o, scal_o, cmax_o):
        xv = x[...]
        xh = jnp.dot(xv, wh_t[...], preferred_element_type=jnp.float32)
        xt = jnp.dot(xv, wt_t[...], preferred_element_type=jnp.float32)
        scal = (jnp.dot(xh, a2[...], preferred_element_type=jnp.float32)
                + jnp.dot(xt, b2[...], preferred_element_type=jnp.float32)
                + jnp.dot(xv, c2[...], preferred_element_type=jnp.float32))
        col = lax.broadcasted_iota(jnp.int32, (NP, WH), 1)
        xh_o[...] = jnp.where(col == 100, 1.0, xh)
        xt_o[...] = jnp.where(col == 100, 1.0, xt)
        scal_o[...] = scal
        cmax_o[...] = jnp.broadcast_to(
            jnp.max(scal, axis=0, keepdims=True), (8, WH))

    return pl.pallas_call(
        body,
        out_shape=[jax.ShapeDtypeStruct((NP, WH), jnp.float32),
                   jax.ShapeDtypeStruct((NP, WH), jnp.float32),
                   jax.ShapeDtypeStruct((NP, WH), jnp.float32),
                   jax.ShapeDtypeStruct((8, WH), jnp.float32)])


def _t_rel_finish():
    def body(acca, accb, grmat, t2_o, cmax_o):
        sa = acca[:, 100:101] + 1e-16
        sb = accb[:, 100:101] + 1e-16
        xr = acca[...] / sa + accb[...] / sb
        t2 = jnp.dot(xr, grmat[...], preferred_element_type=jnp.float32)
        t2_o[...] = t2
        cmax_o[...] = jnp.broadcast_to(
            jnp.max(t2, axis=0, keepdims=True), (8, WH))

    return pl.pallas_call(
        body,
        out_shape=[jax.ShapeDtypeStruct((NRELP, WH), jnp.float32),
                   jax.ShapeDtypeStruct((8, WH), jnp.float32)])


def _t_uv():
    def body(xc, gmat, uv_o, cmax_o):
        uv = jnp.dot(xc[...], gmat[...], preferred_element_type=jnp.float32)
        uv_o[...] = uv
        cmax_o[...] = jnp.broadcast_to(
            jnp.max(uv, axis=0, keepdims=True), (8, WH))

    return pl.pallas_call(
        body,
        out_shape=[jax.ShapeDtypeStruct((NP, WH), jnp.float32),
                   jax.ShapeDtypeStruct((8, WH), jnp.float32)])


def _t_gat_finish():
    blk = 2048
    grid = (NP // blk,)

    def body(gacc, xc, out):
        s = gacc[:, 300:301] + 1e-16
        agg = _lrelu(gacc[:, 0:300] / s)
        o = jnp.concatenate(
            [9.0 * xc[:, 0:600], 9.0 * agg,
             jnp.zeros((blk, W900 - 900), jnp.float32)], axis=1)
        out[...] = o

    return pl.pallas_call(
        body, grid=grid,
        in_specs=[pl.BlockSpec((blk, W300), lambda b: (b, 0)),
                  pl.BlockSpec((blk, W600), lambda b: (b, 0))],
        out_specs=pl.BlockSpec((blk, W900), lambda b: (b, 0)),
        out_shape=jax.ShapeDtypeStruct((NP, W900), jnp.float32))


def _t_lstm1():
    blk = 1024
    grid = (NP // blk,)
    GW = 1024

    def cell(xt, h, c, wf, uf, bf):
        g = (jnp.dot(xt, wf, preferred_element_type=jnp.float32) + bf
             + jnp.dot(h, uf, preferred_element_type=jnp.float32))
        ii = jax.nn.sigmoid(g[:, 0:256])
        ff = jax.nn.sigmoid(g[:, 256:512])
        gg = jnp.tanh(g[:, 512:768])
        oo = jax.nn.sigmoid(g[:, 768:1024])
        c = ff * c + ii * gg
        h = oo * jnp.tanh(c)
        return h, c

    def body(xs, wff, uff, bff, wfr, ufr, bfr, out):
        z = jnp.zeros((blk, 256), jnp.float32)
        h, c = z, z
        for t in range(3):
            h, c = cell(xs[t], h, c, wff[...], uff[...], bff[...])
            out[0, t] = h
        h, c = z, z
        for t in (2, 1, 0):
            h, c = cell(xs[t], h, c, wfr[...], ufr[...], bfr[...])
            out[1, t] = h

    return pl.pallas_call(
        body, grid=grid,
        in_specs=[pl.BlockSpec((3, blk, W300), lambda b: (0, b, 0)),
                  pl.BlockSpec((W300, GW), lambda b: (0, 0)),
                  pl.BlockSpec((256, GW), lambda b: (0, 0)),
                  pl.BlockSpec((1, GW), lambda b: (0, 0)),
                  pl.BlockSpec((W300, GW), lambda b: (0, 0)),
                  pl.BlockSpec((256, GW), lambda b: (0, 0)),
                  pl.BlockSpec((1, GW), lambda b: (0, 0))],
        out_specs=pl.BlockSpec((2, 3, blk, 256), lambda b: (0, 0, b, 0)),
        out_shape=jax.ShapeDtypeStruct((2, 3, NP, 256), jnp.float32))


def _t_lstm2():
    blk = 1024
    grid = (NP // blk,)
    GW = 1536

    def body(xs, wf, uf, bf, out):
        z = jnp.zeros((blk, 384), jnp.float32)
        h, c = z, z
        for t in range(3):
            g = (jnp.dot(xs[t], wf[...], preferred_element_type=jnp.float32)
                 + bf[...]
                 + jnp.dot(h, uf[...], preferred_element_type=jnp.float32))
            ii = jax.nn.sigmoid(g[:, 0:384])
            ff = jax.nn.sigmoid(g[:, 384:768])
            gg = jnp.tanh(g[:, 768:1152])
            oo = jax.nn.sigmoid(g[:, 1152:1536])
            c = ff * c + ii * gg
            h = oo * jnp.tanh(c)
            out[t] = h

    return pl.pallas_call(
        body, grid=grid,
        in_specs=[pl.BlockSpec((3, blk, W300), lambda b: (0, b, 0)),
                  pl.BlockSpec((W300, GW), lambda b: (0, 0)),
                  pl.BlockSpec((384, GW), lambda b: (0, 0)),
                  pl.BlockSpec((1, GW), lambda b: (0, 0))],
        out_specs=pl.BlockSpec((3, blk, 384), lambda b: (0, b, 0)),
        out_shape=jax.ShapeDtypeStruct((3, NP, 384), jnp.float32))


def _t_attn():
    nblk = 512
    grid = (3, NP // nblk)

    def body(q, kt, v, out, m_s, l_s, acc_s):
        n = pl.program_id(1)

        @pl.when(n == 0)
        def _():
            m_s[...] = jnp.full((256, 128), -1e30, jnp.float32)
            l_s[...] = jnp.zeros((256, 128), jnp.float32)
            acc_s[...] = jnp.zeros((256, W900), jnp.float32)

        s = jnp.dot(q[0], kt[0], preferred_element_type=jnp.float32)
        col = lax.broadcasted_iota(jnp.int32, (256, nblk), 1) + n * nblk
        s = jnp.where(col < 10000, s, -1e30)
        m_prev = m_s[:, 0:1]
        m_new = jnp.maximum(m_prev, jnp.max(s, axis=1, keepdims=True))
        p = jnp.exp(s - m_new)
        corr = jnp.exp(m_prev - m_new)
        l_new = l_s[:, 0:1] * corr + jnp.sum(p, axis=1, keepdims=True)
        acc_s[...] = acc_s[...] * corr + jnp.dot(
            p, v[0], preferred_element_type=jnp.float32)
        m_s[...] = jnp.broadcast_to(m_new, (256, 128))
        l_s[...] = jnp.broadcast_to(l_new, (256, 128))

        @pl.when(n == grid[1] - 1)
        def _():
            out[0] = acc_s[...] / l_s[:, 0:1] + q[0]

    return pl.pallas_call(
        body, grid=grid,
        in_specs=[pl.BlockSpec((1, 256, W900), lambda c, n: (c, 0, 0)),
                  pl.BlockSpec((1, W900, nblk), lambda c, n: (c, 0, n)),
                  pl.BlockSpec((1, nblk, W900), lambda c, n: (c, n, 0))],
        out_specs=pl.BlockSpec((1, 256, W900), lambda c, n: (c, 0, 0)),
        out_shape=jax.ShapeDtypeStruct((3, 256, W900), jnp.float32),
        scratch_shapes=[pltpu.VMEM((256, 128), jnp.float32),
                        pltpu.VMEM((256, 128), jnp.float32),
                        pltpu.VMEM((256, W900), jnp.float32)])


def _t_mk():
    def body(re_all, w1t, b1, w2t, b2, out):
        for g in range(2):
            zs = []
            for ch in range(3):
                h = jnp.maximum(
                    jnp.dot(re_all[g, ch], w1t[...],
                            preferred_element_type=jnp.float32) + b1[...],
                    0.0)
                z = jax.nn.sigmoid(
                    jnp.dot(h, w2t[...],
                            preferred_element_type=jnp.float32) + b2[...])
                zs.append(z[:, 0:1])
            m = jnp.maximum(jnp.maximum(zs[0], zs[1]), zs[2])
            es = [jnp.exp(z - m) for z in zs]
            tot = es[0] + es[1] + es[2]
            out[g, 0] = jnp.broadcast_to(es[1] / tot, (256, 128))
            out[g, 1] = jnp.broadcast_to(es[2] / tot, (256, 128))

    return pl.pallas_call(
        body,
        out_shape=jax.ShapeDtypeStruct((2, 2, 256, 128), jnp.float32))


def _t_final_mul():
    blk = 2048
    grid = (4, NP // blk)

    def body(x, f, out):
        out[...] = x[...] * f[:, :, 0:1]

    return pl.pallas_call(
        body, grid=grid,
        in_specs=[pl.BlockSpec((1, blk, W900), lambda g, b: (g, b, 0)),
                  pl.BlockSpec((1, blk, 128), lambda g, b: (g, b, 0))],
        out_specs=pl.BlockSpec((1, blk, W900), lambda g, b: (g, b, 0)),
        out_shape=jax.ShapeDtypeStruct((4, NP, W900), jnp.float32))


def _pad2(a, rows, cols):
    return jnp.pad(a, ((0, rows - a.shape[0]), (0, cols - a.shape[1])))


def _pad1(a, n):
    return jnp.pad(a, (0, n - a.shape[0]))


def _split_tab(a):
    f = a.shape[1] // 2
    return jnp.stack([a[:, :f], a[:, f:]])


def _packT(w, rows, cols, col_map):
    m = jnp.zeros((rows, cols), jnp.float32)
    for mat, off in col_map:
        m = m.at[:mat.shape[1], off:off + mat.shape[0]].set(mat.T)
    return m


def _cvec(x):
    return jnp.full((16,), x, jnp.float32)


def _branch(p, x_e, ei, rel, eia, x_name, wk):
    j_all, i_all = eia[0], eia[1]
    h_e, t_e = ei[0], ei[1]
    jg2 = j_all.reshape(-1, 64)
    ig2 = i_all.reshape(-1, 64)
    hh2 = h_e.reshape(-1, 64)
    tt2 = t_e.reshape(-1, 64)
    rr2 = rel.reshape(-1, 64)

    deg2 = wk["deg"](i_all)
    dinv = wk["dinv"](deg2)
    dinv128 = jnp.broadcast_to(dinv[0][:, None], (NP, 128))

    xp = _pad2(x_e, NP, W300)
    y = wk["prescale"](xp, dinv128)
    graw = wk["edge320u"](_split_tab(y), jg2, ig2)
    graw = jnp.concatenate([graw[0], graw[1]], axis=1)
    x1, y1 = wk["highway_p"](xp, graw, dinv128, wk["hw1T"])
    graw = wk["edge320u"](_split_tab(y1), jg2, ig2)
    graw = jnp.concatenate([graw[0], graw[1]], axis=1)
    x2 = wk["highway"](x1, graw, dinv128, wk["hw2T"])

    xh_aug, xt_aug, scal, cmax = wk["proj"](x2, wk["whT"], wk["wtT"],
                                            wk["a2"], wk["b2"], wk["c2"])
    pcol = jnp.asarray(scal[:, 0]).copy()
    qcol = jnp.asarray(scal[:, 1]).copy()
    rcol = jnp.asarray(scal[:, 2]).copy()
    s2col = jnp.asarray(scal[:, 3]).copy()
    ehcol = jnp.asarray(scal[:, 4]).copy()
    c1 = jnp.maximum(cmax[0, 0] + cmax[0, 1], 0.0)
    c2c = jnp.maximum(cmax[0, 2] + cmax[0, 3], 0.0)
    wA = wk["wpass_nn"](pcol, qcol, h_e, t_e, _cvec(c1))
    accA = wk["edge128r"](_split_tab(xh_aug), hh2, rr2, wA)
    accA = jnp.concatenate([accA[0], accA[1]], axis=1)
    wB = wk["wpass_nn"](rcol, s2col, h_e, t_e, _cvec(c2c))
    accB = wk["edge128r"](_split_tab(xt_aug), tt2, rr2, wB)
    accB = jnp.concatenate([accB[0], accB[1]], axis=1)
    t2, cmax_t2 = wk["rel_finish"](accA, accB, wk["grT"])
    t2col = jnp.asarray(t2[:, 0]).copy()

    cr = jnp.maximum(cmax[0, 4] + cmax_t2[0, 0], 0.0)
    wR = wk["wpass_nr"](ehcol, t2col, h_e, rel, _cvec(cr))
    x2_aug = jnp.concatenate(
        [x2[:, 0:300],
         jnp.ones((NP, 1), jnp.float32),
         jnp.zeros((NP, W300 - 301), jnp.float32)], axis=1)
    accR = wk["edge320r"](_split_tab(x2_aug), tt2, rr2, wR)
    accR = jnp.concatenate([accR[0], accR[1]], axis=1)
    sR = accR[:, 300:301] + 1e-16
    gat_r_rows = accR / sR
    gat_r_full = jnp.zeros((NP, 300), jnp.float32).at[:NRELP].set(
        gat_r_rows[:, 0:300])

    x_cat = jnp.concatenate(
        [x2[:, 0:300], gat_r_full,
         jnp.zeros((NP, W600 - 600), jnp.float32)], axis=1)
    uv, cmax_uv = wk["uv"](x_cat, wk["gmat"])
    ucol = jnp.asarray(uv[:, 0]).copy()
    vcol = jnp.asarray(uv[:, 1]).copy()
    cg = jnp.maximum(cmax_uv[0, 0] + cmax_uv[0, 1], 0.0)
    wG = wk["wpass_nn"](ucol, vcol, i_all, j_all, _cvec(cg))
    accG = wk["edge320"](_split_tab(x2_aug), jg2, ig2, wG)
    accG = jnp.concatenate([accG[0], accG[1]], axis=1)
    xf = wk["gat_finish"](accG, x_cat)

    xs = jnp.stack([_pad2(x_name[:, 300 * t:300 * (t + 1)], NP, W300)
                    for t in range(3)])
    return xf, xs


def kernel(x_e1, edge_index1, rel1, edge_index_all1, rel_all1, x_name1,
           onehot1, x_e2, edge_index2, rel2, edge_index_all2, rel_all2,
           x_name2, onehot2, data_batch, params):
    p = params
    wk = {}
    wk["deg"] = _make_deg_pass()
    wk["dinv"] = _t_dinv()
    wk["prescale"] = _t_prescale()
    wk["edge320"] = _make_edge_pass(NP, 160, True)
    wk["edge320u"] = _make_edge_pass(NP, 160, False)
    wk["edge128r"] = _make_edge_pass(NRELP, 64, True)
    wk["edge320r"] = _make_edge_pass(NRELP, 160, True)
    wk["wpass_nn"] = _make_weight_pass(NP, NP)
    wk["wpass_nr"] = _make_weight_pass(NP, NRELP)
    wk["highway_p"] = _t_highway(True)
    wk["highway"] = _t_highway(False)
    wk["proj"] = _t_proj()
    wk["rel_finish"] = _t_rel_finish()
    wk["uv"] = _t_uv()
    wk["gat_finish"] = _t_gat_finish()
    wk["lstm1"] = _t_lstm1()
    wk["lstm2"] = _t_lstm2()
    wk["attn"] = _t_attn()
    wk["mk"] = _t_mk()
    wk["fmul"] = _t_final_mul()
    wk["bgather"] = _make_bgather()
    wk["factor"] = _make_factor_pass()

    wk["hw1T"] = _pad2(p["hw1_W"].T, W300, W300)
    wk["hw2T"] = _pad2(p["hw2_W"].T, W300, W300)
    wk["whT"] = _pad2(p["ge_wh"].T, W300, WH)
    wk["wtT"] = _pad2(p["ge_wt"].T, W300, WH)
    wk["a2"] = _packT(None, WH, WH, [(p["ge_ah1"], 0), (p["ge_at1"], 2)])
    wk["b2"] = _packT(None, WH, WH, [(p["ge_ah2"], 1), (p["ge_at2"], 3)])
    wk["c2"] = _packT(None, W300, WH, [(p["gr_ah"], 4)])
    wk["grT"] = _packT(None, WH, WH, [(p["gr_ar"], 0)])
    wk["gmat"] = _packT(None, W600, WH, [(p["g_ai"], 0), (p["g_aj"], 1)])

    def lstm1_w(pre):
        w = jnp.zeros((W300, 1024), jnp.float32)
        u = jnp.zeros((256, 1024), jnp.float32)
        b = jnp.zeros((1, 1024), jnp.float32)
        wih, whh = p[pre + "_Wih"], p[pre + "_Whh"]
        bih, bhh = p[pre + "_bih"], p[pre + "_bhh"]
        for g in range(4):
            w = w.at[:300, 256 * g:256 * g + 150].set(
                wih[150 * g:150 * (g + 1)].T)
            u = u.at[:150, 256 * g:256 * g + 150].set(
                whh[150 * g:150 * (g + 1)].T)
            b = b.at[0, 256 * g:256 * g + 150].set(
                bih[150 * g:150 * (g + 1)] + bhh[150 * g:150 * (g + 1)])
        return w, u, b

    l1f = lstm1_w("l1f")
    l1r = lstm1_w("l1r")

    w2 = jnp.zeros((W300, 1536), jnp.float32)
    u2 = jnp.zeros((384, 1536), jnp.float32)
    b2l = jnp.zeros((1, 1536), jnp.float32)
    for g in range(4):
        w2 = w2.at[:300, 384 * g:384 * g + 300].set(
            p["l2_Wih"][300 * g:300 * (g + 1)].T)
        u2 = u2.at[:300, 384 * g:384 * g + 300].set(
            p["l2_Whh"][300 * g:300 * (g + 1)].T)
        b2l = b2l.at[0, 384 * g:384 * g + 300].set(
            p["l2_bih"][300 * g:300 * (g + 1)]
            + p["l2_bhh"][300 * g:300 * (g + 1)])

    xf1, xs1 = _branch(p, x_e1, edge_index1, rel1, edge_index_all1, x_name1,
                       wk)
    xf2, xs2 = _branch(p, x_e2, edge_index2, rel2, edge_index_all2, x_name2,
                       wk)

    def lstms(xs, onehot):
        o1raw = wk["lstm1"](xs, *l1f, *l1r)
        o1 = jnp.concatenate(
            [o1raw[d, t][:, :150] for t in range(3) for d in range(2)],
            axis=1)
        o1 = _pad2(o1, NP, W900)
        xo = jnp.stack([_pad2(onehot[:, 300 * t:300 * (t + 1)], NP, W300)
                        for t in range(3)])
        o2raw = wk["lstm2"](xo, w2, u2, b2l)
        o2 = jnp.concatenate([o2raw[t][:, :300] for t in range(3)], axis=1)
        o2 = _pad2(o2, NP, W900)
        return o1, o2

    o11, o12 = lstms(xs1, onehot1)
    o21, o22 = lstms(xs2, onehot2)

    b0 = jnp.asarray(data_batch[:, 0]).copy()
    b1 = jnp.asarray(data_batch[:, 1]).copy()
    q1 = wk["bgather"](xf1, o11, o12, b0)
    q2 = wk["bgather"](xf2, o21, o22, b1)

    kv1 = jnp.stack([xf2, o21, o22])
    kv2 = jnp.stack([xf1, o11, o12])
    kt1 = jnp.transpose(kv1, (0, 2, 1))
    kt2 = jnp.transpose(kv2, (0, 2, 1))
    re1 = wk["attn"](q1, kt1, kv1)
    re2 = wk["attn"](q2, kt2, kv2)

    mk_w1t = _pad2(p["mk_W1"].T, W900, 512)
    mk_b1 = _pad1(p["mk_b1"], 512)[None, :]
    mk_w2t = _pad2(p["mk_W2"].T, 512, 128)
    mk_b2 = _pad1(p["mk_b2"], 128)[None, :]
    kg = wk["mk"](jnp.stack([re1, re2]), mk_w1t, mk_b1, mk_w2t, mk_b2)
    vals = jnp.stack([kg[0, 0, :, 0], kg[0, 1, :, 0],
                      kg[1, 0, :, 0], kg[1, 1, :, 0]])
    factors = wk["factor"](jnp.stack([b0, b1]), vals)
    fb = jnp.broadcast_to(factors[:, :, None], (4, NP, 128))
    stack4 = jnp.stack([o11, o12, o21, o22])
    mod = wk["fmul"](stack4, fb)

    def cut(a):
        return a[:10000, :900]

    return (cut(xf1), cut(mod[0]), cut(mod[1]),
            cut(xf2), cut(mod[2]), cut(mod[3]))


# speedup vs baseline: 7.5415x; 1.0295x over previous
import functools
import jax
import jax.numpy as jnp
from jax import lax
from jax.experimental import pallas as pl
from jax.experimental.pallas import tpu as pltpu
from jax.experimental.pallas import tpu_sc as plsc

NC, NS = 2, 16
NP = 10240
E = 160000
NRELP = 512
W300 = 320
W600 = 640
W900 = 912
WH = 128

_SC_PARAMS = pltpu.CompilerParams(needs_layout_passes=False,
                                  use_tc_tiling_on_sc=False)
_MESH = dict(core_axis_name="c", subcore_axis_name="s", num_cores=NC,
             num_subcores=NS)


def _make_deg_pass():
    per_w = E // (NC * NS)
    mesh = plsc.VectorSubcoreMesh(**_MESH)
    scratch = dict(
        acc=pltpu.VMEM((NP,), jnp.float32),
        iv=pltpu.VMEM((per_w,), jnp.int32),
        obuf=pltpu.VMEM((NP // NS,), jnp.float32),
        rbuf=pltpu.VMEM((NS, NP // NS), jnp.float32),
        shr=pltpu.VMEM_SHARED((NS, NP), jnp.float32),
    )

    def body(idx, out, sc):
        c = lax.axis_index("c")
        s = lax.axis_index("s")
        w = s * NC + c
        z16 = jnp.zeros((16,), jnp.float32)
        ones = jnp.ones((16,), jnp.float32)
        miota = lax.iota(jnp.int32, 16)

        def zrow(r, _):
            sc["acc"][pl.ds(r * 16, 16)] = z16
            return 0

        lax.fori_loop(0, NP // 16, zrow, 0)
        pltpu.sync_copy(idx.at[pl.ds(w * per_w, per_w)], sc["iv"])

        def step(g, _):
            i16 = sc["iv"][pl.ds(g * 16, 16)]
            for l in range(16):
                plsc.addupdate_scatter(sc["acc"], [i16], ones,
                                       mask=miota == l)
            return 0

        lax.fori_loop(0, per_w // 16, step, 0)
        rem = per_w % 16
        if rem:
            i16 = sc["iv"][pl.ds(per_w - 16, 16)]
            for l in range(16 - rem, 16):
                plsc.addupdate_scatter(sc["acc"], [i16], ones,
                                       mask=miota == l)
        pltpu.sync_copy(sc["acc"], sc["shr"].at[s])
        plsc.subcore_barrier()
        cols = NP // NS
        pltpu.sync_copy(sc["shr"].at[:, pl.ds(s * cols, cols)], sc["rbuf"])
        for f in range(cols // 16):
            v = jnp.zeros((16,), jnp.float32)
            for r in range(NS):
                v = v + sc["rbuf"][r, pl.ds(f * 16, 16)]
            sc["obuf"][pl.ds(f * 16, 16)] = v
        pltpu.sync_copy(sc["obuf"], out.at[c].at[pl.ds(s * cols, cols)])

    return pl.kernel(body, out_type=jax.ShapeDtypeStruct((NC, NP),
                                                         jnp.float32),
                     mesh=mesh, scratch_types=[scratch],
                     compiler_params=_SC_PARAMS)


def _make_weight_pass(na, nb):
    per_w = E // (NC * NS)
    mesh = plsc.VectorSubcoreMesh(**_MESH)
    scratch = dict(
        ta=pltpu.VMEM((na,), jnp.float32),
        tb=pltpu.VMEM((nb,), jnp.float32),
        av=pltpu.VMEM((per_w,), jnp.int32),
        bv=pltpu.VMEM((per_w,), jnp.int32),
        wv=pltpu.VMEM((per_w,), jnp.float32),
        cv=pltpu.VMEM((16,), jnp.float32),
    )

    def body(tab_a, tab_b, idx_a, idx_b, cvec, out, sc):
        c = lax.axis_index("c")
        s = lax.axis_index("s")
        w = s * NC + c
        base = w * per_w
        pltpu.sync_copy(tab_a, sc["ta"])
        pltpu.sync_copy(tab_b, sc["tb"])
        pltpu.sync_copy(cvec, sc["cv"])
        pltpu.sync_copy(idx_a.at[pl.ds(base, per_w)], sc["av"])
        pltpu.sync_copy(idx_b.at[pl.ds(base, per_w)], sc["bv"])
        cval = sc["cv"][:]

        def compute(off):
            a16 = plsc.load_gather(sc["ta"], [sc["av"][pl.ds(off, 16)]])
            b16 = plsc.load_gather(sc["tb"], [sc["bv"][pl.ds(off, 16)]])
            e = a16 + b16
            t = jnp.where(e >= 0, e, 0.01 * e)
            sc["wv"][pl.ds(off, 16)] = jnp.exp(t - cval)

        def step(g, _):
            compute(g * 16)
            return 0

        lax.fori_loop(0, per_w // 16, step, 0)
        if per_w % 16:
            compute(per_w - 16)
        pltpu.sync_copy(sc["wv"], out.at[pl.ds(base, per_w)])

    return pl.kernel(body, out_type=jax.ShapeDtypeStruct((E,), jnp.float32),
                     mesh=mesh, scratch_types=[scratch],
                     compiler_params=_SC_PARAMS)


def _make_edge_pass(n_dst, fc, weighted):
    n_sc = E // 1280
    base_n, extra = divmod(n_sc, NS)
    rows_t = n_dst // NS
    mesh = plsc.VectorSubcoreMesh(**_MESH)

    scratch = dict(
        acc=pltpu.VMEM_SHARED((n_dst, fc), jnp.float32),
        ab0=pltpu.VMEM((64, fc), jnp.float32),
        ab1=pltpu.VMEM((64, fc), jnp.float32),
        jv=pltpu.VMEM((20, 64), jnp.int32),
        iv=pltpu.VMEM((20, 64), jnp.int32),
        g0=pltpu.SemaphoreType.DMA,
        g1=pltpu.SemaphoreType.DMA,
        s0=pltpu.SemaphoreType.DMA,
        s1=pltpu.SemaphoreType.DMA,
    )
    if weighted:
        scratch["wv"] = pltpu.VMEM((1296,), jnp.float32)

    def body(*refs):
        if weighted:
            (table, idx_g, idx_s, wref, out, sc) = refs
        else:
            (table, idx_g, idx_s, out, sc) = refs
        c = lax.axis_index("c")
        s = lax.axis_index("s")

        z16 = jnp.zeros((16,), jnp.float32)

        def zrow(r, _):
            for f in range(fc // 16):
                sc["ab0"][r, pl.ds(16 * f, 16)] = z16
            return 0

        lax.fori_loop(0, 64, zrow, 0)
        nz_full, nz_rem = divmod(rows_t, 64)
        for rchunk in range(nz_full):
            pltpu.sync_copy(
                sc["ab0"],
                sc["acc"].at[pl.ds(s * rows_t + rchunk * 64, 64), :])
        if nz_rem:
            pltpu.sync_copy(
                sc["ab0"].at[pl.ds(0, nz_rem), :],
                sc["acc"].at[pl.ds(s * rows_t + nz_full * 64, nz_rem), :])
        plsc.subcore_barrier()

        n_my = base_n + jnp.where(s < extra, 1, 0)
        start = s * base_n + jnp.minimum(s, extra)

        def scale(buf, woff, _r0=None):
            def scale_row(r, _):
                w = jnp.full((16,), sc["wv"][pl.ds(woff + r, 16)][0],
                             jnp.float32)
                for f in range(fc // 16):
                    buf[r, pl.ds(16 * f, 16)] = (
                        buf[r, pl.ds(16 * f, 16)] * w)
                return 0

            lax.fori_loop(0, 64, scale_row, 0)

        def sc_loop(u, _):
            srow = (start + u) * 20
            sbase = (start + u) * 1280
            pltpu.sync_copy(idx_g.at[pl.ds(srow, 20), :], sc["jv"])
            pltpu.sync_copy(idx_s.at[pl.ds(srow, 20), :], sc["iv"])
            if weighted:
                pltpu.sync_copy(wref.at[pl.ds(sbase, 1280)],
                                sc["wv"].at[pl.ds(0, 1280)])

            def drain(buf, sem):
                pltpu.make_async_copy(buf, sc["acc"].at[sc["iv"].at[0]],
                                      sem).wait()

            def pair(t, _):
                @pl.when(t > 0)
                def _():
                    drain(sc["ab0"], sc["s0"])

                ga = pltpu.async_copy(table.at[c].at[sc["jv"].at[2 * t]],
                                      sc["ab0"], sc["g0"])

                @pl.when(t > 0)
                def _():
                    drain(sc["ab1"], sc["s1"])

                gb = pltpu.async_copy(table.at[c].at[sc["jv"].at[2 * t + 1]],
                                      sc["ab1"], sc["g1"])
                ga.wait()
                if weighted:
                    scale(sc["ab0"], 128 * t)
                pltpu.async_copy(sc["ab0"],
                                 sc["acc"].at[sc["iv"].at[2 * t]],
                                 sc["s0"], add=True)
                gb.wait()
                if weighted:
                    scale(sc["ab1"], 128 * t + 64)
                pltpu.async_copy(sc["ab1"],
                                 sc["acc"].at[sc["iv"].at[2 * t + 1]],
                                 sc["s1"], add=True)
                return 0

            lax.fori_loop(0, 10, pair, 0)
            drain(sc["ab0"], sc["s0"])
            drain(sc["ab1"], sc["s1"])
            return 0

        lax.fori_loop(0, n_my, sc_loop, 0)
        plsc.subcore_barrier()
        pltpu.sync_copy(sc["acc"].at[pl.ds(s * rows_t, rows_t), :],
                        out.at[c].at[pl.ds(s * rows_t, rows_t), :])

    return pl.kernel(body,
                     out_type=jax.ShapeDtypeStruct((NC, n_dst, fc),
                                                   jnp.float32),
                     mesh=mesh, scratch_types=[scratch],
                     compiler_params=_SC_PARAMS)


def _make_bgather():
    B = 256
    per_w = B // (NC * NS)
    mesh = plsc.VectorSubcoreMesh(**_MESH)
    scratch = dict(
        iv=pltpu.VMEM((per_w,), jnp.int32),
        rbuf=pltpu.VMEM((per_w, W900), jnp.float32),
        sem=pltpu.SemaphoreType.DMA,
    )

    def body(t0, t1, t2, idx, out, sc):
        c = lax.axis_index("c")
        s = lax.axis_index("s")
        w = s * NC + c
        pltpu.sync_copy(idx.at[pl.ds(w * per_w, per_w)], sc["iv"])
        for t, tab in enumerate((t0, t1, t2)):
            pltpu.async_copy(tab.at[sc["iv"]], sc["rbuf"], sc["sem"]).wait()
            pltpu.sync_copy(sc["rbuf"],
                            out.at[t].at[pl.ds(w * per_w, per_w), :])

    return pl.kernel(body,
                     out_type=jax.ShapeDtypeStruct((3, 256, W900),
                                                   jnp.float32),
                     mesh=mesh, scratch_types=[scratch],
                     compiler_params=_SC_PARAMS)


def _make_factor_pass():
    B = 256
    mesh = plsc.VectorSubcoreMesh(**_MESH)
    scratch = dict(
        fv=pltpu.VMEM((NP,), jnp.float32),
        iv=pltpu.VMEM((B,), jnp.int32),
        vv=pltpu.VMEM((B,), jnp.float32),
    )

    def body(idxs, vals, out, sc):
        c = lax.axis_index("c")
        s = lax.axis_index("s")
        w = s * NC + c
        miota = lax.iota(jnp.int32, 16)

        @pl.when(w < 4)
        def _():
            ones = jnp.ones((16,), jnp.float32)

            def orow(r, _):
                sc["fv"][pl.ds(r * 16, 16)] = ones
                return 0

            lax.fori_loop(0, NP // 16, orow, 0)
            pltpu.sync_copy(idxs.at[w // 2], sc["iv"])
            pltpu.sync_copy(vals.at[w], sc["vv"])

            def step(g, _):
                i16 = sc["iv"][pl.ds(g * 16, 16)]
                v16 = sc["vv"][pl.ds(g * 16, 16)]
                for l in range(16):
                    plsc.store_scatter(sc["fv"], [i16], v16,
                                       mask=miota == l)
                return 0

            lax.fori_loop(0, B // 16, step, 0)
            pltpu.sync_copy(sc["fv"], out.at[w])

    return pl.kernel(body, out_type=jax.ShapeDtypeStruct((4, NP),
                                                         jnp.float32),
                     mesh=mesh, scratch_types=[scratch],
                     compiler_params=_SC_PARAMS)


def _lrelu(x):
    return jnp.where(x >= 0, x, 0.01 * x)


def _t_dinv():
    def body(deg2, out):
        deg = deg2[0:1, :] + deg2[1:2, :]
        dinv = lax.rsqrt(deg)
        idx = lax.broadcasted_iota(jnp.int32, (1, NP), 1)
        dinv = jnp.where(idx < 10000, dinv, 0.0)
        out[...] = jnp.broadcast_to(dinv, (8, NP))

    return pl.pallas_call(body, out_shape=jax.ShapeDtypeStruct((8, NP),
                                                               jnp.float32))


def _t_prescale():
    def body(x, dinv, out):
        out[...] = x[...] * dinv[:, 0:1]

    return pl.pallas_call(body, out_shape=jax.ShapeDtypeStruct((NP, W300),
                                                               jnp.float32))


def _t_highway(prescale):
    blk = 2048
    grid = (NP // blk,)

    def body(x, graw, dinv, wt, out, yout=None):
        x2 = jnp.maximum(graw[...] * dinv[:, 0:1], 0.0)
        gate = _lrelu(jnp.dot(x[...], wt[...],
                              preferred_element_type=jnp.float32) + 1e-08)
        o = _lrelu(gate * x2 + (1.0 - gate) * x[...])
        out[...] = o
        if prescale:
            yout[...] = o * dinv[:, 0:1]

    rowspec = pl.BlockSpec((blk, W300), lambda b: (b, 0))
    if prescale:
        out_shape = [jax.ShapeDtypeStruct((NP, W300), jnp.float32),
                     jax.ShapeDtypeStruct((NP, W300), jnp.float32)]
        out_specs = [rowspec, rowspec]
    else:
        out_shape = jax.ShapeDtypeStruct((NP, W300), jnp.float32)
        out_specs = rowspec

    return pl.pallas_call(
        body, grid=grid,
        in_specs=[rowspec,
                  rowspec,
                  pl.BlockSpec((blk, 128), lambda b: (b, 0)),
                  pl.BlockSpec((W300, W300), lambda b: (0, 0))],
        out_specs=out_specs, out_shape=out_shape)


def _t_proj():
    def body(x, wh_t, wt_t, a2, b2, c2, xh_o, xt_o, scal_o, cmax_o):
        xv = x[...]
        xh = jnp.dot(xv, wh_t[...], preferred_element_type=jnp.float32)
        xt = jnp.dot(xv, wt_t[...], preferred_element_type=jnp.float32)
        scal = (jnp.dot(xh, a2[...], preferred_element_type=jnp.float32)
                + jnp.dot(xt, b2[...], preferred_element_type=jnp.float32)
                + jnp.dot(xv, c2[...], preferred_element_type=jnp.float32))
        col = lax.broadcasted_iota(jnp.int32, (NP, WH), 1)
        xh_o[...] = jnp.where(col == 100, 1.0, xh)
        xt_o[...] = jnp.where(col == 100, 1.0, xt)
        scal_o[...] = scal
        cmax_o[...] = jnp.broadcast_to(
            jnp.max(scal, axis=0, keepdims=True), (8, WH))

    return pl.pallas_call(
        body,
        out_shape=[jax.ShapeDtypeStruct((NP, WH), jnp.float32),
                   jax.ShapeDtypeStruct((NP, WH), jnp.float32),
                   jax.ShapeDtypeStruct((NP, WH), jnp.float32),
                   jax.ShapeDtypeStruct((8, WH), jnp.float32)])


def _t_rel_finish():
    def body(acca, accb, grmat, t2_o, cmax_o):
        sa = acca[:, 100:101] + 1e-16
        sb = accb[:, 100:101] + 1e-16
        xr = acca[...] / sa + accb[...] / sb
        t2 = jnp.dot(xr, grmat[...], preferred_element_type=jnp.float32)
        t2_o[...] = t2
        cmax_o[...] = jnp.broadcast_to(
            jnp.max(t2, axis=0, keepdims=True), (8, WH))

    return pl.pallas_call(
        body,
        out_shape=[jax.ShapeDtypeStruct((NRELP, WH), jnp.float32),
                   jax.ShapeDtypeStruct((8, WH), jnp.float32)])


def _t_uv():
    def body(xc, gmat, uv_o, cmax_o):
        uv = jnp.dot(xc[...], gmat[...], preferred_element_type=jnp.float32)
        uv_o[...] = uv
        cmax_o[...] = jnp.broadcast_to(
            jnp.max(uv, axis=0, keepdims=True), (8, WH))

    return pl.pallas_call(
        body,
        out_shape=[jax.ShapeDtypeStruct((NP, WH), jnp.float32),
                   jax.ShapeDtypeStruct((8, WH), jnp.float32)])


def _t_gat_finish():
    blk = 2048
    grid = (NP // blk,)

    def body(gacc, xc, out):
        s = gacc[:, 300:301] + 1e-16
        agg = _lrelu(gacc[:, 0:300] / s)
        o = jnp.concatenate(
            [9.0 * xc[:, 0:600], 9.0 * agg,
             jnp.zeros((blk, W900 - 900), jnp.float32)], axis=1)
        out[...] = o

    return pl.pallas_call(
        body, grid=grid,
        in_specs=[pl.BlockSpec((blk, W300), lambda b: (b, 0)),
                  pl.BlockSpec((blk, W600), lambda b: (b, 0))],
        out_specs=pl.BlockSpec((blk, W900), lambda b: (b, 0)),
        out_shape=jax.ShapeDtypeStruct((NP, W900), jnp.float32))


def _t_lstm1():
    blk = 1024
    grid = (NP // blk,)
    GW = 1024

    def cell(xt, h, c, wf, uf, bf):
        g = (jnp.dot(xt, wf, preferred_element_type=jnp.float32) + bf
             + jnp.dot(h, uf, preferred_element_type=jnp.float32))
        ii = jax.nn.sigmoid(g[:, 0:256])
        ff = jax.nn.sigmoid(g[:, 256:512])
        gg = jnp.tanh(g[:, 512:768])
        oo = jax.nn.sigmoid(g[:, 768:1024])
        c = ff * c + ii * gg
        h = oo * jnp.tanh(c)
        return h, c

    def body(xs, wff, uff, bff, wfr, ufr, bfr, out):
        z = jnp.zeros((blk, 256), jnp.float32)
        h, c = z, z
        for t in range(3):
            h, c = cell(xs[t], h, c, wff[...], uff[...], bff[...])
            out[0, t] = h
        h, c = z, z
        for t in (2, 1, 0):
            h, c = cell(xs[t], h, c, wfr[...], ufr[...], bfr[...])
            out[1, t] = h

    return pl.pallas_call(
        body, grid=grid,
        in_specs=[pl.BlockSpec((3, blk, W300), lambda b: (0, b, 0)),
                  pl.BlockSpec((W300, GW), lambda b: (0, 0)),
                  pl.BlockSpec((256, GW), lambda b: (0, 0)),
                  pl.BlockSpec((1, GW), lambda b: (0, 0)),
                  pl.BlockSpec((W300, GW), lambda b: (0, 0)),
                  pl.BlockSpec((256, GW), lambda b: (0, 0)),
                  pl.BlockSpec((1, GW), lambda b: (0, 0))],
        out_specs=pl.BlockSpec((2, 3, blk, 256), lambda b: (0, 0, b, 0)),
        out_shape=jax.ShapeDtypeStruct((2, 3, NP, 256), jnp.float32))


def _t_lstm2():
    blk = 1024
    grid = (NP // blk,)
    GW = 1536

    def body(xs, wf, uf, bf, out):
        z = jnp.zeros((blk, 384), jnp.float32)
        h, c = z, z
        for t in range(3):
            g = (jnp.dot(xs[t], wf[...], preferred_element_type=jnp.float32)
                 + bf[...]
                 + jnp.dot(h, uf[...], preferred_element_type=jnp.float32))
            ii = jax.nn.sigmoid(g[:, 0:384])
            ff = jax.nn.sigmoid(g[:, 384:768])
            gg = jnp.tanh(g[:, 768:1152])
            oo = jax.nn.sigmoid(g[:, 1152:1536])
            c = ff * c + ii * gg
            h = oo * jnp.tanh(c)
            out[t] = h

    return pl.pallas_call(
        body, grid=grid,
        in_specs=[pl.BlockSpec((3, blk, W300), lambda b: (0, b, 0)),
                  pl.BlockSpec((W300, GW), lambda b: (0, 0)),
                  pl.BlockSpec((384, GW), lambda b: (0, 0)),
                  pl.BlockSpec((1, GW), lambda b: (0, 0))],
        out_specs=pl.BlockSpec((3, blk, 384), lambda b: (0, b, 0)),
        out_shape=jax.ShapeDtypeStruct((3, NP, 384), jnp.float32))


def _t_attn():
    nblk = 512
    grid = (3, NP // nblk)

    def body(q, kt, v, out, m_s, l_s, acc_s):
        n = pl.program_id(1)

        @pl.when(n == 0)
        def _():
            m_s[...] = jnp.full((256, 128), -1e30, jnp.float32)
            l_s[...] = jnp.zeros((256, 128), jnp.float32)
            acc_s[...] = jnp.zeros((256, W900), jnp.float32)

        s = jnp.dot(q[0], kt[0], preferred_element_type=jnp.float32)
        col = lax.broadcasted_iota(jnp.int32, (256, nblk), 1) + n * nblk
        s = jnp.where(col < 10000, s, -1e30)
        m_prev = m_s[:, 0:1]
        m_new = jnp.maximum(m_prev, jnp.max(s, axis=1, keepdims=True))
        p = jnp.exp(s - m_new)
        corr = jnp.exp(m_prev - m_new)
        l_new = l_s[:, 0:1] * corr + jnp.sum(p, axis=1, keepdims=True)
        acc_s[...] = acc_s[...] * corr + jnp.dot(
            p, v[0], preferred_element_type=jnp.float32)
        m_s[...] = jnp.broadcast_to(m_new, (256, 128))
        l_s[...] = jnp.broadcast_to(l_new, (256, 128))

        @pl.when(n == grid[1] - 1)
        def _():
            out[0] = acc_s[...] / l_s[:, 0:1] + q[0]

    return pl.pallas_call(
        body, grid=grid,
        in_specs=[pl.BlockSpec((1, 256, W900), lambda c, n: (c, 0, 0)),
                  pl.BlockSpec((1, W900, nblk), lambda c, n: (c, 0, n)),
                  pl.BlockSpec((1, nblk, W900), lambda c, n: (c, n, 0))],
        out_specs=pl.BlockSpec((1, 256, W900), lambda c, n: (c, 0, 0)),
        out_shape=jax.ShapeDtypeStruct((3, 256, W900), jnp.float32),
        scratch_shapes=[pltpu.VMEM((256, 128), jnp.float32),
                        pltpu.VMEM((256, 128), jnp.float32),
                        pltpu.VMEM((256, W900), jnp.float32)])


def _t_mk():
    def body(re_all, w1t, b1, w2t, b2, out):
        for g in range(2):
            zs = []
            for ch in range(3):
                h = jnp.maximum(
                    jnp.dot(re_all[g, ch], w1t[...],
                            preferred_element_type=jnp.float32) + b1[...],
                    0.0)
                z = jax.nn.sigmoid(
                    jnp.dot(h, w2t[...],
                            preferred_element_type=jnp.float32) + b2[...])
                zs.append(z[:, 0:1])
            m = jnp.maximum(jnp.maximum(zs[0], zs[1]), zs[2])
            es = [jnp.exp(z - m) for z in zs]
            tot = es[0] + es[1] + es[2]
            out[g, 0] = jnp.broadcast_to(es[1] / tot, (256, 128))
            out[g, 1] = jnp.broadcast_to(es[2] / tot, (256, 128))

    return pl.pallas_call(
        body,
        out_shape=jax.ShapeDtypeStruct((2, 2, 256, 128), jnp.float32))


def _t_final_mul():
    blk = 2048
    grid = (4, NP // blk)

    def body(x, f, out):
        out[...] = x[...] * f[:, :, 0:1]

    return pl.pallas_call(
        body, grid=grid,
        in_specs=[pl.BlockSpec((1, blk, W900), lambda g, b: (g, b, 0)),
                  pl.BlockSpec((1, blk, 128), lambda g, b: (g, b, 0))],
        out_specs=pl.BlockSpec((1, blk, W900), lambda g, b: (g, b, 0)),
        out_shape=jax.ShapeDtypeStruct((4, NP, W900), jnp.float32))


def _pad2(a, rows, cols):
    return jnp.pad(a, ((0, rows - a.shape[0]), (0, cols - a.shape[1])))


def _pad1(a, n):
    return jnp.pad(a, (0, n - a.shape[0]))


def _split_tab(a):
    f = a.shape[1] // 2
    return jnp.stack([a[:, :f], a[:, f:]])


def _packT(w, rows, cols, col_map):
    m = jnp.zeros((rows, cols), jnp.float32)
    for mat, off in col_map:
        m = m.at[:mat.shape[1], off:off + mat.shape[0]].set(mat.T)
    return m


def _cvec(x):
    return jnp.full((16,), x, jnp.float32)


def _branch(p, x_e, ei, rel, eia, x_name, wk):
    j_all, i_all = eia[0], eia[1]
    h_e, t_e = ei[0], ei[1]
    jg2 = j_all.reshape(-1, 64)
    ig2 = i_all.reshape(-1, 64)
    hh2 = h_e.reshape(-1, 64)
    tt2 = t_e.reshape(-1, 64)
    rr2 = rel.reshape(-1, 64)

    deg2 = wk["deg"](i_all)
    dinv = wk["dinv"](deg2)
    dinv128 = jnp.broadcast_to(dinv[0][:, None], (NP, 128))

    xp = _pad2(x_e, NP, W300)
    y = wk["prescale"](xp, dinv128)
    graw = wk["edge320u"](_split_tab(y), jg2, ig2)
    graw = jnp.concatenate([graw[0], graw[1]], axis=1)
    x1, y1 = wk["highway_p"](xp, graw, dinv128, wk["hw1T"])
    graw = wk["edge320u"](_split_tab(y1), jg2, ig2)
    graw = jnp.concatenate([graw[0], graw[1]], axis=1)
    x2 = wk["highway"](x1, graw, dinv128, wk["hw2T"])

    xh_aug, xt_aug, scal, cmax = wk["proj"](x2, wk["whT"], wk["wtT"],
                                            wk["a2"], wk["b2"], wk["c2"])
    pcol = jnp.asarray(scal[:, 0]).copy()
    qcol = jnp.asarray(scal[:, 1]).copy()
    rcol = jnp.asarray(scal[:, 2]).copy()
    s2col = jnp.asarray(scal[:, 3]).copy()
    ehcol = jnp.asarray(scal[:, 4]).copy()
    c1 = jnp.maximum(cmax[0, 0] + cmax[0, 1], 0.0)
    c2c = jnp.maximum(cmax[0, 2] + cmax[0, 3], 0.0)
    wA = wk["wpass_nn"](pcol, qcol, h_e, t_e, _cvec(c1))
    accA = wk["edge128r"](_split_tab(xh_aug), hh2, rr2, wA)
    accA = jnp.concatenate([accA[0], accA[1]], axis=1)
    wB = wk["wpass_nn"](rcol, s2col, h_e, t_e, _cvec(c2c))
    accB = wk["edge128r"](_split_tab(xt_aug), tt2, rr2, wB)
    accB = jnp.concatenate([accB[0], accB[1]], axis=1)
    t2, cmax_t2 = wk["rel_finish"](accA, accB, wk["grT"])
    t2col = jnp.asarray(t2[:, 0]).copy()

    cr = jnp.maximum(cmax[0, 4] + cmax_t2[0, 0], 0.0)
    wR = wk["wpass_nr"](ehcol, t2col, h_e, rel, _cvec(cr))
    x2_aug = jnp.concatenate(
        [x2[:, 0:300],
         jnp.ones((NP, 1), jnp.float32),
         jnp.zeros((NP, W300 - 301), jnp.float32)], axis=1)
    accR = wk["edge320r"](_split_tab(x2_aug), tt2, rr2, wR)
    accR = jnp.concatenate([accR[0], accR[1]], axis=1)
    sR = accR[:, 300:301] + 1e-16
    gat_r_rows = accR / sR
    gat_r_full = jnp.zeros((NP, 300), jnp.float32).at[:NRELP].set(
        gat_r_rows[:, 0:300])

    x_cat = jnp.concatenate(
        [x2[:, 0:300], gat_r_full,
         jnp.zeros((NP, W600 - 600), jnp.float32)], axis=1)
    uv, cmax_uv = wk["uv"](x_cat, wk["gmat"])
    ucol = jnp.asarray(uv[:, 0]).copy()
    vcol = jnp.asarray(uv[:, 1]).copy()
    cg = jnp.maximum(cmax_uv[0, 0] + cmax_uv[0, 1], 0.0)
    wG = wk["wpass_nn"](ucol, vcol, i_all, j_all, _cvec(cg))
    accG = wk["edge320"](_split_tab(x2_aug), jg2, ig2, wG)
    accG = jnp.concatenate([accG[0], accG[1]], axis=1)
    xf = wk["gat_finish"](accG, x_cat)

    xs = jnp.stack([_pad2(x_name[:, 300 * t:300 * (t + 1)], NP, W300)
                    for t in range(3)])
    return xf, xs


def kernel(x_e1, edge_index1, rel1, edge_index_all1, rel_all1, x_name1,
           onehot1, x_e2, edge_index2, rel2, edge_index_all2, rel_all2,
           x_name2, onehot2, data_batch, params):
    p = params
    wk = {}
    wk["deg"] = _make_deg_pass()
    wk["dinv"] = _t_dinv()
    wk["prescale"] = _t_prescale()
    wk["edge320"] = _make_edge_pass(NP, 160, True)
    wk["edge320u"] = _make_edge_pass(NP, 160, False)
    wk["edge128r"] = _make_edge_pass(NRELP, 64, True)
    wk["edge320r"] = _make_edge_pass(NRELP, 160, True)
    wk["wpass_nn"] = _make_weight_pass(NP, NP)
    wk["wpass_nr"] = _make_weight_pass(NP, NRELP)
    wk["highway_p"] = _t_highway(True)
    wk["highway"] = _t_highway(False)
    wk["proj"] = _t_proj()
    wk["rel_finish"] = _t_rel_finish()
    wk["uv"] = _t_uv()
    wk["gat_finish"] = _t_gat_finish()
    wk["lstm1"] = _t_lstm1()
    wk["lstm2"] = _t_lstm2()
    wk["attn"] = _t_attn()
    wk["mk"] = _t_mk()
    wk["fmul"] = _t_final_mul()
    wk["bgather"] = _make_bgather()
    wk["factor"] = _make_factor_pass()

    wk["hw1T"] = _pad2(p["hw1_W"].T, W300, W300)
    wk["hw2T"] = _pad2(p["hw2_W"].T, W300, W300)
    wk["whT"] = _pad2(p["ge_wh"].T, W300, WH)
    wk["wtT"] = _pad2(p["ge_wt"].T, W300, WH)
    wk["a2"] = _packT(None, WH, WH, [(p["ge_ah1"], 0), (p["ge_at1"], 2)])
    wk["b2"] = _packT(None, WH, WH, [(p["ge_ah2"], 1), (p["ge_at2"], 3)])
    wk["c2"] = _packT(None, W300, WH, [(p["gr_ah"], 4)])
    wk["grT"] = _packT(None, WH, WH, [(p["gr_ar"], 0)])
    wk["gmat"] = _packT(None, W600, WH, [(p["g_ai"], 0), (p["g_aj"], 1)])

    def lstm1_w(pre):
        w = jnp.zeros((W300, 1024), jnp.float32)
        u = jnp.zeros((256, 1024), jnp.float32)
        b = jnp.zeros((1, 1024), jnp.float32)
        wih, whh = p[pre + "_Wih"], p[pre + "_Whh"]
        bih, bhh = p[pre + "_bih"], p[pre + "_bhh"]
        for g in range(4):
            w = w.at[:300, 256 * g:256 * g + 150].set(
                wih[150 * g:150 * (g + 1)].T)
            u = u.at[:150, 256 * g:256 * g + 150].set(
                whh[150 * g:150 * (g + 1)].T)
            b = b.at[0, 256 * g:256 * g + 150].set(
                bih[150 * g:150 * (g + 1)] + bhh[150 * g:150 * (g + 1)])
        return w, u, b

    l1f = lstm1_w("l1f")
    l1r = lstm1_w("l1r")

    w2 = jnp.zeros((W300, 1536), jnp.float32)
    u2 = jnp.zeros((384, 1536), jnp.float32)
    b2l = jnp.zeros((1, 1536), jnp.float32)
    for g in range(4):
        w2 = w2.at[:300, 384 * g:384 * g + 300].set(
            p["l2_Wih"][300 * g:300 * (g + 1)].T)
        u2 = u2.at[:300, 384 * g:384 * g + 300].set(
            p["l2_Whh"][300 * g:300 * (g + 1)].T)
        b2l = b2l.at[0, 384 * g:384 * g + 300].set(
            p["l2_bih"][300 * g:300 * (g + 1)]
            + p["l2_bhh"][300 * g:300 * (g + 1)])

    xf1, xs1 = _branch(p, x_e1, edge_index1, rel1, edge_index_all1, x_name1,
                       wk)
    xf2, xs2 = _branch(p, x_e2, edge_index2, rel2, edge_index_all2, x_name2,
                       wk)

    def lstms(xs, onehot):
        o1raw = wk["lstm1"](xs, *l1f, *l1r)
        o1 = jnp.concatenate(
            [o1raw[d, t][:, :150] for t in range(3) for d in range(2)],
            axis=1)
        o1 = _pad2(o1, NP, W900)
        xo = jnp.stack([_pad2(onehot[:, 300 * t:300 * (t + 1)], NP, W300)
                        for t in range(3)])
        o2raw = wk["lstm2"](xo, w2, u2, b2l)
        o2 = jnp.concatenate([o2raw[t][:, :300] for t in range(3)], axis=1)
        o2 = _pad2(o2, NP, W900)
        return o1, o2

    o11, o12 = lstms(xs1, onehot1)
    o21, o22 = lstms(xs2, onehot2)

    b0 = jnp.asarray(data_batch[:, 0]).copy()
    b1 = jnp.asarray(data_batch[:, 1]).copy()
    q1 = wk["bgather"](xf1, o11, o12, b0)
    q2 = wk["bgather"](xf2, o21, o22, b1)

    kv1 = jnp.stack([xf2, o21, o22])
    kv2 = jnp.stack([xf1, o11, o12])
    kt1 = jnp.transpose(kv1, (0, 2, 1))
    kt2 = jnp.transpose(kv2, (0, 2, 1))
    re1 = wk["attn"](q1, kt1, kv1)
    re2 = wk["attn"](q2, kt2, kv2)

    mk_w1t = _pad2(p["mk_W1"].T, W900, 512)
    mk_b1 = _pad1(p["mk_b1"], 512)[None, :]
    mk_w2t = _pad2(p["mk_W2"].T, 512, 128)
    mk_b2 = _pad1(p["mk_b2"], 128)[None, :]
    kg = wk["mk"](jnp.stack([re1, re2]), mk_w1t, mk_b1, mk_w2t, mk_b2)
    vals = jnp.stack([kg[0, 0, :, 0], kg[0, 1, :, 0],
                      kg[1, 0, :, 0], kg[1, 1, :, 0]])
    factors = wk["factor"](jnp.stack([b0, b1]), vals)
    fb = jnp.broadcast_to(factors[:, :, None], (4, NP, 128))
    stack4 = jnp.stack([o11, o12, o21, o22])
    mod = wk["fmul"](stack4, fb)

    def cut(a):
        return a[:10000, :900]

    return (cut(xf1), cut(mod[0]), cut(mod[1]),
            cut(xf2), cut(mod[2]), cut(mod[3]))
